# Initial kernel scaffold; baseline (speedup 1.0000x reference)
import jax
import jax.numpy as jnp
from jax.experimental import pallas as pl


def kernel(x, edge_index, W_gat, att_src, att_dst, b_gat, W_pool, b_pool, W_emb, b_emb, W_i1, b_i1, W_i2, b_i2):
    raise NotImplementedError("write your pallas kernel here")


# speedup vs baseline: 42.6038x; 42.6038x over previous
import functools

import jax
import jax.numpy as jnp
from jax import lax
from jax.experimental import pallas as pl
from jax.experimental.pallas import tpu as pltpu
from jax.experimental.pallas import tpu_sc as plsc

N = 10000
E = 320000
D_IN = 128
H = 8
C = 128
HC = H * C
D_EMB = 32
HE = H * D_EMB
D_HID = 128

NC = 2
NS = 16
NW = NC * NS
K = 128
E_TOT = E + N
CH = -(-E_TOT // (NW * K))
E_PAD = NW * K * CH
NP = 10112
NP_T = NP // NS

BLK = 1000


def _tc_a_body(x_ref, wg_ref, bd_ref, asrc_ref, adst_ref,
               h_ref, he_ref, as_ref, ad_ref, mx_ref):
    i = pl.program_id(0)
    xb = x_ref[...]
    hb = jnp.dot(xb, wg_ref[...], preferred_element_type=jnp.float32)
    h_ref[...] = hb
    he_ref[...] = jnp.dot(hb, bd_ref[...], preferred_element_type=jnp.float32)
    zs = jnp.zeros((BLK, 8), dtype=jnp.float32)
    ts = hb * asrc_ref[...]
    td = hb * adst_ref[...]
    a_s = jnp.concatenate(
        [jnp.sum(ts[:, hh * C:(hh + 1) * C], axis=1, keepdims=True)
         for hh in range(H)] + [zs], axis=1)
    a_d = jnp.concatenate(
        [jnp.sum(td[:, hh * C:(hh + 1) * C], axis=1, keepdims=True)
         for hh in range(H)] + [zs], axis=1)
    as_ref[...] = a_s
    ad_ref[...] = a_d
    ms = jnp.max(a_s, axis=0, keepdims=True)
    md = jnp.max(a_d, axis=0, keepdims=True)
    upd = jnp.concatenate([ms, md, jnp.full((6, 16), -jnp.inf, jnp.float32)],
                          axis=0)

    @pl.when(i == 0)
    def _():
        mx_ref[...] = upd

    @pl.when(i > 0)
    def _():
        mx_ref[...] = jnp.maximum(mx_ref[...], upd)


def _tc_a(x, w_gat, bd, asrc_flat, adst_flat):
    grid = N // BLK
    return pl.pallas_call(
        _tc_a_body,
        grid=(grid,),
        in_specs=[
            pl.BlockSpec((BLK, D_IN), lambda i: (i, 0)),
            pl.BlockSpec((D_IN, HC), lambda i: (0, 0)),
            pl.BlockSpec((HC, HE), lambda i: (0, 0)),
            pl.BlockSpec((1, HC), lambda i: (0, 0)),
            pl.BlockSpec((1, HC), lambda i: (0, 0)),
        ],
        out_specs=[
            pl.BlockSpec((BLK, HC), lambda i: (i, 0)),
            pl.BlockSpec((BLK, HE), lambda i: (i, 0)),
            pl.BlockSpec((BLK, 16), lambda i: (i, 0)),
            pl.BlockSpec((BLK, 16), lambda i: (i, 0)),
            pl.BlockSpec((8, 16), lambda i: (0, 0)),
        ],
        out_shape=[
            jax.ShapeDtypeStruct((N, HC), jnp.float32),
            jax.ShapeDtypeStruct((N, HE), jnp.float32),
            jax.ShapeDtypeStruct((N, 16), jnp.float32),
            jax.ShapeDtypeStruct((N, 16), jnp.float32),
            jax.ShapeDtypeStruct((8, 16), jnp.float32),
        ],
    )(x, w_gat, bd, asrc_flat, adst_flat)


def _leaky(t):
    return jnp.where(t >= 0.0, t, 0.2 * t)


def _sc1_body(src_hbm, dst_hbm, as_hbm, ad_hbm, m_hbm, dpart_hbm,
              den_sh, src_v, dst_v, as_v, ad_v, s_v, z_v, m_v,
              sem0, sem1, sem2):
    cid = lax.axis_index("c")
    sid = lax.axis_index("s")
    wid = cid * NS + sid

    @pl.loop(0, NP_T)
    def _(r):
        z_v[r] = jnp.zeros((16,), jnp.float32)
    pltpu.sync_copy(z_v, den_sh.at[pl.ds(sid * NP_T, NP_T)])
    pltpu.sync_copy(m_hbm, m_v)
    plsc.subcore_barrier()

    mvec = m_v[...]
    tile_base = wid * (CH * K)

    @pl.loop(0, CH)
    def _(j):
        base = tile_base + j * K
        pltpu.async_copy(src_hbm.at[pl.ds(base, K)], src_v, sem0)
        pltpu.async_copy(dst_hbm.at[pl.ds(base, K)], dst_v, sem1).wait()
        pltpu.make_async_copy(src_hbm.at[pl.ds(base, K)], src_v, sem0).wait()
        pltpu.async_copy(as_hbm.at[src_v], as_v, sem0)
        pltpu.async_copy(ad_hbm.at[dst_v], ad_v, sem1)
        pltpu.make_async_copy(as_hbm.at[src_v], as_v, sem0).wait()
        pltpu.make_async_copy(ad_hbm.at[dst_v], ad_v, sem1).wait()

        @pl.loop(0, K)
        def _(r):
            t = _leaky(as_v[r] + ad_v[r]) - mvec
            s_v[r] = jnp.exp(t)

        pltpu.sync_copy(s_v, den_sh.at[dst_v], add=True)

    plsc.subcore_barrier()
    pltpu.sync_copy(den_sh.at[pl.ds(sid * NP_T, NP_T)],
                    dpart_hbm.at[cid, pl.ds(sid * NP_T, NP_T)])


def _sc1(src, dst, a_s, a_d, m16):
    mesh = plsc.VectorSubcoreMesh(core_axis_name="c", subcore_axis_name="s", num_cores=NC, num_subcores=NS)
    f = pl.kernel(
        _sc1_body,
        out_type=jax.ShapeDtypeStruct((NC, NP, 16), jnp.float32),
        mesh=mesh,
        compiler_params=pltpu.CompilerParams(use_tc_tiling_on_sc=False),
        scratch_types=[
            pltpu.VMEM_SHARED((NP, 16), jnp.float32),
            pltpu.VMEM((K,), jnp.int32),
            pltpu.VMEM((K,), jnp.int32),
            pltpu.VMEM((K, 16), jnp.float32),
            pltpu.VMEM((K, 16), jnp.float32),
            pltpu.VMEM((K, 16), jnp.float32),
            pltpu.VMEM((NP_T, 16), jnp.float32),
            pltpu.VMEM((16,), jnp.float32),
            pltpu.SemaphoreType.DMA,
            pltpu.SemaphoreType.DMA,
            pltpu.SemaphoreType.DMA,
        ],
    )
    return f(src, dst, a_s, a_d, m16)


def _sc2_body(src_hbm, dst_hbm, as_hbm, ad_hbm, dn_hbm, he_hbm, m_hbm,
              epart_hbm, wpart_hbm,
              emb_sh, w_sh, src_v, dst_v, as_v, ad_v, dn_v, he_v,
              al_v, msg_v, z_v, z16_v, m_v, sem0, sem1, sem2, sem3):
    cid = lax.axis_index("c")
    sid = lax.axis_index("s")
    wid = cid * NS + sid

    @pl.loop(0, NP_T)
    def _(r):
        z_v[r] = jnp.zeros((32,), jnp.float32)
        z16_v[r] = jnp.zeros((16,), jnp.float32)
    pltpu.sync_copy(z_v, emb_sh.at[pl.ds(sid * NP_T, NP_T)])
    pltpu.sync_copy(z16_v, w_sh.at[pl.ds(sid * NP_T, NP_T)])
    pltpu.sync_copy(m_hbm, m_v)
    plsc.subcore_barrier()

    mvec = m_v[...]
    tile_base = wid * (CH * K)

    @pl.loop(0, CH)
    def _(j):
        base = tile_base + j * K
        pltpu.async_copy(src_hbm.at[pl.ds(base, K)], src_v, sem0)
        pltpu.async_copy(dst_hbm.at[pl.ds(base, K)], dst_v, sem1).wait()
        pltpu.make_async_copy(src_hbm.at[pl.ds(base, K)], src_v, sem0).wait()
        pltpu.async_copy(as_hbm.at[src_v], as_v, sem0)
        pltpu.async_copy(ad_hbm.at[dst_v], ad_v, sem1)
        pltpu.async_copy(dn_hbm.at[dst_v], dn_v, sem2)
        pltpu.async_copy(he_hbm.at[src_v], he_v, sem3)
        pltpu.make_async_copy(as_hbm.at[src_v], as_v, sem0).wait()
        pltpu.make_async_copy(ad_hbm.at[dst_v], ad_v, sem1).wait()
        pltpu.make_async_copy(dn_hbm.at[dst_v], dn_v, sem2).wait()
        pltpu.make_async_copy(he_hbm.at[src_v], he_v, sem3).wait()

        @pl.loop(0, K)
        def _(r):
            t = _leaky(as_v[r] + ad_v[r]) - mvec
            al = jnp.exp(t) / dn_v[r]
            al_v[r] = al
            acc0 = jnp.zeros((16,), jnp.float32)
            acc1 = jnp.zeros((16,), jnp.float32)
            for hh in range(H):
                a = al[hh]
                acc0 = acc0 + a * he_v[r, pl.ds(hh * 32, 16)]
                acc1 = acc1 + a * he_v[r, pl.ds(hh * 32 + 16, 16)]
            msg_v[r, pl.ds(0, 16)] = acc0
            msg_v[r, pl.ds(16, 16)] = acc1

        pltpu.sync_copy(msg_v, emb_sh.at[dst_v], add=True)
        pltpu.sync_copy(al_v, w_sh.at[src_v], add=True)

    plsc.subcore_barrier()
    pltpu.sync_copy(emb_sh.at[pl.ds(sid * NP_T, NP_T)],
                    epart_hbm.at[cid, pl.ds(sid * NP_T, NP_T)])
    pltpu.sync_copy(w_sh.at[pl.ds(sid * NP_T, NP_T)],
                    wpart_hbm.at[cid, pl.ds(sid * NP_T, NP_T)])


def _sc2(src, dst, a_s, a_d, denom, he, m16):
    mesh = plsc.VectorSubcoreMesh(core_axis_name="c", subcore_axis_name="s", num_cores=NC, num_subcores=NS)
    f = pl.kernel(
        _sc2_body,
        out_type=(jax.ShapeDtypeStruct((NC, NP, 32), jnp.float32),
                  jax.ShapeDtypeStruct((NC, NP, 16), jnp.float32)),
        mesh=mesh,
        compiler_params=pltpu.CompilerParams(use_tc_tiling_on_sc=False),
        scratch_types=[
            pltpu.VMEM_SHARED((NP, 32), jnp.float32),
            pltpu.VMEM_SHARED((NP, 16), jnp.float32),
            pltpu.VMEM((K,), jnp.int32),
            pltpu.VMEM((K,), jnp.int32),
            pltpu.VMEM((K, 16), jnp.float32),
            pltpu.VMEM((K, 16), jnp.float32),
            pltpu.VMEM((K, 16), jnp.float32),
            pltpu.VMEM((K, HE), jnp.float32),
            pltpu.VMEM((K, 16), jnp.float32),
            pltpu.VMEM((K, 32), jnp.float32),
            pltpu.VMEM((NP_T, 32), jnp.float32),
            pltpu.VMEM((NP_T, 16), jnp.float32),
            pltpu.VMEM((16,), jnp.float32),
            pltpu.SemaphoreType.DMA,
            pltpu.SemaphoreType.DMA,
            pltpu.SemaphoreType.DMA,
            pltpu.SemaphoreType.DMA,
        ],
    )
    return f(src, dst, a_s, a_d, denom, he, m16)


def _tc_b1_body(h_ref, wp_ref, p16_ref, wpool_ref, bg_ref, bp_ref,
                g_ref, acc_ref):
    i = pl.program_id(0)
    w_blk = wp_ref[0] + wp_ref[1]
    wexp = jnp.dot(w_blk, p16_ref[...],
                   preferred_element_type=jnp.float32)
    part = jnp.sum(h_ref[...] * wexp, axis=0, keepdims=True)

    @pl.when(i == 0)
    def _():
        acc_ref[...] = part

    @pl.when(i > 0)
    def _():
        acc_ref[...] = acc_ref[...] + part

    @pl.when(i == pl.num_programs(0) - 1)
    def _():
        m = acc_ref[...] / jnp.float32(N) + bg_ref[...]
        g_ref[...] = jnp.maximum(
            jnp.dot(m, wpool_ref[...], preferred_element_type=jnp.float32)
            + bp_ref[...], 0.0)


def _tc_b1(h, wpart, p16, w_pool, b_gat, b_pool):
    grid = N // BLK
    return pl.pallas_call(
        _tc_b1_body,
        grid=(grid,),
        in_specs=[
            pl.BlockSpec((BLK, HC), lambda i: (i, 0)),
            pl.BlockSpec((NC, BLK, 16), lambda i: (0, i, 0)),
            pl.BlockSpec((16, HC), lambda i: (0, 0)),
            pl.BlockSpec((HC, D_HID), lambda i: (0, 0)),
            pl.BlockSpec((1, HC), lambda i: (0, 0)),
            pl.BlockSpec((1, D_HID), lambda i: (0, 0)),
        ],
        out_specs=pl.BlockSpec((1, D_HID), lambda i: (0, 0)),
        out_shape=jax.ShapeDtypeStruct((1, D_HID), jnp.float32),
        scratch_shapes=[pltpu.VMEM((1, HC), jnp.float32)],
    )(h, wpart, p16, w_pool, b_gat, b_pool)


def _tc_b2_body(ep_ref, ebias_ref, g_ref, wi1_ref, bi1_ref, wi2_ref, bi2_ref,
                emb_ref, inf_ref):
    emb = ep_ref[0] + ep_ref[1] + ebias_ref[...]
    emb_ref[...] = emb
    gb = jnp.broadcast_to(g_ref[...], (BLK, D_HID))
    comb = jnp.concatenate([emb, gb], axis=1)
    hid = jnp.maximum(
        jnp.dot(comb, wi1_ref[...], preferred_element_type=jnp.float32)
        + bi1_ref[...], 0.0)
    z = jnp.dot(hid, wi2_ref[...], preferred_element_type=jnp.float32) \
        + bi2_ref[...]
    inf_ref[...] = 1.0 / (1.0 + jnp.exp(-z))


def _tc_b2(epart, ebias, g, w_i1, b_i1, w_i2, b_i2):
    grid = N // BLK
    return pl.pallas_call(
        _tc_b2_body,
        grid=(grid,),
        in_specs=[
            pl.BlockSpec((NC, BLK, D_EMB), lambda i: (0, i, 0)),
            pl.BlockSpec((1, D_EMB), lambda i: (0, 0)),
            pl.BlockSpec((1, D_HID), lambda i: (0, 0)),
            pl.BlockSpec((D_EMB + D_HID, 64), lambda i: (0, 0)),
            pl.BlockSpec((1, 64), lambda i: (0, 0)),
            pl.BlockSpec((64, 1), lambda i: (0, 0)),
            pl.BlockSpec((1, 1), lambda i: (0, 0)),
        ],
        out_specs=[
            pl.BlockSpec((BLK, D_EMB), lambda i: (i, 0)),
            pl.BlockSpec((BLK, 1), lambda i: (i, 0)),
        ],
        out_shape=[
            jax.ShapeDtypeStruct((N, D_EMB), jnp.float32),
            jax.ShapeDtypeStruct((N, 1), jnp.float32),
        ],
    )(epart, ebias, g, w_i1, b_i1, w_i2, b_i2)


def kernel(x, edge_index, W_gat, att_src, att_dst, b_gat, W_pool, b_pool,
           W_emb, b_emb, W_i1, b_i1, W_i2, b_i2):
    asrc_flat = att_src.reshape(1, HC)
    adst_flat = att_dst.reshape(1, HC)
    bd = jnp.zeros((H, C, H, D_EMB), jnp.float32)
    ii = jnp.arange(H)
    bd = bd.at[ii, :, ii, :].set(W_emb.reshape(H, C, D_EMB))
    bd = bd.reshape(HC, HE)
    p16 = jnp.zeros((16, H, C), jnp.float32)
    p16 = p16.at[jnp.arange(H), jnp.arange(H), :].set(1.0)
    p16 = p16.reshape(16, HC)

    h, he, a_s, a_d, mx = _tc_a(x, W_gat, bd, asrc_flat, adst_flat)
    m16 = _leaky(mx[0] + mx[1])

    loop_idx = jnp.arange(N, dtype=jnp.int32)
    src = jnp.concatenate([edge_index[0].astype(jnp.int32), loop_idx,
                           jnp.full((E_PAD - E_TOT,), N, jnp.int32)])
    dst = jnp.concatenate([edge_index[1].astype(jnp.int32), loop_idx,
                           jnp.full((E_PAD - E_TOT,), N, jnp.int32)])

    pad_rows = NP - N
    a_s_p = jnp.concatenate([a_s, jnp.zeros((pad_rows, 16), jnp.float32)])
    a_d_p = jnp.concatenate([a_d, jnp.zeros((pad_rows, 16), jnp.float32)])
    he_p = jnp.concatenate([he, jnp.zeros((pad_rows, HE), jnp.float32)])

    dpart = _sc1(src, dst, a_s_p, a_d_p, m16)
    denom = dpart[0] + dpart[1]

    epart, wpart = _sc2(src, dst, a_s_p, a_d_p, denom, he_p, m16)

    g = _tc_b1(h, wpart[:, :N, :], p16, W_pool, b_gat.reshape(1, HC),
               b_pool.reshape(1, D_HID))
    ebias = (b_gat @ W_emb + b_emb).reshape(1, D_EMB)
    emb, inf = _tc_b2(epart[:, :N, :], ebias, g, W_i1,
                      b_i1.reshape(1, 64), W_i2, b_i2.reshape(1, 1))
    return (emb, inf)


# speedup vs baseline: 49.3636x; 1.1587x over previous
import functools

import jax
import jax.numpy as jnp
from jax import lax
from jax.experimental import pallas as pl
from jax.experimental.pallas import tpu as pltpu
from jax.experimental.pallas import tpu_sc as plsc

N = 10000
E = 320000
D_IN = 128
H = 8
C = 128
HC = H * C
D_EMB = 32
HE = H * D_EMB
D_HID = 128

NC = 2
NS = 16
NW = NC * NS
K = 128
E_TOT = E + N
CH = 2 * (-(-E_TOT // (NW * K * 2)))
E_PAD = NW * K * CH
NP = 10112
NP_T = NP // NS

BLK = 1000


def _tc_a_body(x_ref, wg_ref, bd_ref, asrc_ref, adst_ref,
               h_ref, he_ref, as_ref, ad_ref, mx_ref):
    i = pl.program_id(0)
    xb = x_ref[...]
    hb = jnp.dot(xb, wg_ref[...], preferred_element_type=jnp.float32)
    h_ref[...] = hb
    he_ref[...] = jnp.dot(hb, bd_ref[...], preferred_element_type=jnp.float32)
    zs = jnp.zeros((BLK, 8), dtype=jnp.float32)
    ts = hb * asrc_ref[...]
    td = hb * adst_ref[...]
    a_s = jnp.concatenate(
        [jnp.sum(ts[:, hh * C:(hh + 1) * C], axis=1, keepdims=True)
         for hh in range(H)] + [zs], axis=1)
    a_d = jnp.concatenate(
        [jnp.sum(td[:, hh * C:(hh + 1) * C], axis=1, keepdims=True)
         for hh in range(H)] + [zs], axis=1)
    as_ref[...] = a_s
    ad_ref[...] = a_d
    ms = jnp.max(a_s, axis=0, keepdims=True)
    md = jnp.max(a_d, axis=0, keepdims=True)
    upd = jnp.concatenate([ms, md, jnp.full((6, 16), -jnp.inf, jnp.float32)],
                          axis=0)

    @pl.when(i == 0)
    def _():
        mx_ref[...] = upd

    @pl.when(i > 0)
    def _():
        mx_ref[...] = jnp.maximum(mx_ref[...], upd)


def _tc_a(x, w_gat, bd, asrc_flat, adst_flat):
    grid = N // BLK
    return pl.pallas_call(
        _tc_a_body,
        grid=(grid,),
        in_specs=[
            pl.BlockSpec((BLK, D_IN), lambda i: (i, 0)),
            pl.BlockSpec((D_IN, HC), lambda i: (0, 0)),
            pl.BlockSpec((HC, HE), lambda i: (0, 0)),
            pl.BlockSpec((1, HC), lambda i: (0, 0)),
            pl.BlockSpec((1, HC), lambda i: (0, 0)),
        ],
        out_specs=[
            pl.BlockSpec((BLK, HC), lambda i: (i, 0)),
            pl.BlockSpec((BLK, HE), lambda i: (i, 0)),
            pl.BlockSpec((BLK, 16), lambda i: (i, 0)),
            pl.BlockSpec((BLK, 16), lambda i: (i, 0)),
            pl.BlockSpec((8, 16), lambda i: (0, 0)),
        ],
        out_shape=[
            jax.ShapeDtypeStruct((N, HC), jnp.float32),
            jax.ShapeDtypeStruct((N, HE), jnp.float32),
            jax.ShapeDtypeStruct((N, 16), jnp.float32),
            jax.ShapeDtypeStruct((N, 16), jnp.float32),
            jax.ShapeDtypeStruct((8, 16), jnp.float32),
        ],
    )(x, w_gat, bd, asrc_flat, adst_flat)


def _leaky(t):
    return jnp.where(t >= 0.0, t, 0.2 * t)


ZCH = NP_T // 4


def _sc1_body(src_hbm, dst_hbm, as_hbm, ad_hbm, m_hbm, dpart_hbm,
              den_sh, src0, dst0, src1, dst1, as0, ad0, as1, ad1,
              s_v, z_v, m_v, semi0, semi1, semg0, semg1):
    cid = lax.axis_index("c")
    sid = lax.axis_index("s")
    wid = cid * NS + sid

    @pl.loop(0, ZCH)
    def _(r):
        z_v[r] = jnp.zeros((16,), jnp.float32)
    for q in range(4):
        pltpu.sync_copy(z_v, den_sh.at[pl.ds(sid * NP_T + q * ZCH, ZCH)])
    pltpu.sync_copy(m_hbm, m_v)
    plsc.subcore_barrier()

    mvec = m_v[...]
    tb = wid * (CH * K)
    idx = ((src0, dst0), (src1, dst1))
    dat = ((as0, ad0), (as1, ad1))
    sgi = (semi0, semi1)
    sgg = (semg0, semg1)

    def fire_idx(j, b):
        pltpu.async_copy(src_hbm.at[pl.ds(tb + j * K, K)], idx[b][0], sgi[b])
        pltpu.async_copy(dst_hbm.at[pl.ds(tb + j * K, K)], idx[b][1], sgi[b])

    def wait_idx(j, b):
        pltpu.make_async_copy(src_hbm.at[pl.ds(tb + j * K, K)], idx[b][0],
                              sgi[b]).wait()
        pltpu.make_async_copy(dst_hbm.at[pl.ds(tb + j * K, K)], idx[b][1],
                              sgi[b]).wait()

    def fire_g(b):
        pltpu.async_copy(as_hbm.at[idx[b][0]], dat[b][0], sgg[b])
        pltpu.async_copy(ad_hbm.at[idx[b][1]], dat[b][1], sgg[b])

    def wait_g(b):
        pltpu.make_async_copy(as_hbm.at[idx[b][0]], dat[b][0], sgg[b]).wait()
        pltpu.make_async_copy(ad_hbm.at[idx[b][1]], dat[b][1], sgg[b]).wait()

    fire_idx(0, 0)
    wait_idx(0, 0)
    fire_g(0)
    fire_idx(1, 1)

    @pl.loop(0, CH // 2)
    def _(jo):
        for b in range(2):
            j = jo * 2 + b
            nb = 1 - b
            wait_g(b)

            @pl.when(j < CH - 1)
            def _():
                wait_idx(j + 1, nb)
                fire_g(nb)

            as_v, ad_v = dat[b]

            @pl.loop(0, K)
            def _(r):
                t = _leaky(as_v[r] + ad_v[r]) - mvec
                s_v[r] = jnp.exp(t)

            pltpu.sync_copy(s_v, den_sh.at[idx[b][1]], add=True)

            @pl.when(j < CH - 2)
            def _():
                fire_idx(j + 2, b)

    plsc.subcore_barrier()
    pltpu.sync_copy(den_sh.at[pl.ds(sid * NP_T, NP_T)],
                    dpart_hbm.at[cid, pl.ds(sid * NP_T, NP_T)])


def _sc1(src, dst, a_s, a_d, m16):
    mesh = plsc.VectorSubcoreMesh(core_axis_name="c", subcore_axis_name="s", num_cores=NC, num_subcores=NS)
    f = pl.kernel(
        _sc1_body,
        out_type=jax.ShapeDtypeStruct((NC, NP, 16), jnp.float32),
        mesh=mesh,
        compiler_params=pltpu.CompilerParams(use_tc_tiling_on_sc=False),
        scratch_types=[
            pltpu.VMEM_SHARED((NP, 16), jnp.float32),
            pltpu.VMEM((K,), jnp.int32),
            pltpu.VMEM((K,), jnp.int32),
            pltpu.VMEM((K,), jnp.int32),
            pltpu.VMEM((K,), jnp.int32),
            pltpu.VMEM((K, 16), jnp.float32),
            pltpu.VMEM((K, 16), jnp.float32),
            pltpu.VMEM((K, 16), jnp.float32),
            pltpu.VMEM((K, 16), jnp.float32),
            pltpu.VMEM((K, 16), jnp.float32),
            pltpu.VMEM((ZCH, 16), jnp.float32),
            pltpu.VMEM((16,), jnp.float32),
            pltpu.SemaphoreType.DMA,
            pltpu.SemaphoreType.DMA,
            pltpu.SemaphoreType.DMA,
            pltpu.SemaphoreType.DMA,
        ],
    )
    return f(src, dst, a_s, a_d, m16)


def _sc2_body(src_hbm, dst_hbm, as_hbm, addn_hbm, he_hbm, m_hbm,
              epart_hbm, wpart_hbm,
              emb_sh, w_sh, src0, dst0, src1, dst1,
              as0, ad0, he0, as1, ad1, he1,
              al_v, msg_v, z_v, z16_v, m_v, semi0, semi1, semg0, semg1):
    cid = lax.axis_index("c")
    sid = lax.axis_index("s")
    wid = cid * NS + sid

    @pl.loop(0, ZCH)
    def _(r):
        z_v[r] = jnp.zeros((32,), jnp.float32)
        z16_v[r] = jnp.zeros((16,), jnp.float32)
    for q in range(4):
        pltpu.sync_copy(z_v, emb_sh.at[pl.ds(sid * NP_T + q * ZCH, ZCH)])
        pltpu.sync_copy(z16_v, w_sh.at[pl.ds(sid * NP_T + q * ZCH, ZCH)])
    pltpu.sync_copy(m_hbm, m_v)
    plsc.subcore_barrier()

    mvec = m_v[...]
    tb = wid * (CH * K)
    idx = ((src0, dst0), (src1, dst1))
    dat = ((as0, ad0, he0), (as1, ad1, he1))
    sgi = (semi0, semi1)
    sgg = (semg0, semg1)

    def fire_idx(j, b):
        pltpu.async_copy(src_hbm.at[pl.ds(tb + j * K, K)], idx[b][0], sgi[b])
        pltpu.async_copy(dst_hbm.at[pl.ds(tb + j * K, K)], idx[b][1], sgi[b])

    def wait_idx(j, b):
        pltpu.make_async_copy(src_hbm.at[pl.ds(tb + j * K, K)], idx[b][0],
                              sgi[b]).wait()
        pltpu.make_async_copy(dst_hbm.at[pl.ds(tb + j * K, K)], idx[b][1],
                              sgi[b]).wait()

    def fire_g(b):
        pltpu.async_copy(as_hbm.at[idx[b][0]], dat[b][0], sgg[b])
        pltpu.async_copy(addn_hbm.at[idx[b][1]], dat[b][1], sgg[b])
        pltpu.async_copy(he_hbm.at[idx[b][0]], dat[b][2], sgg[b])

    def wait_g(b):
        pltpu.make_async_copy(as_hbm.at[idx[b][0]], dat[b][0], sgg[b]).wait()
        pltpu.make_async_copy(addn_hbm.at[idx[b][1]], dat[b][1],
                              sgg[b]).wait()
        pltpu.make_async_copy(he_hbm.at[idx[b][0]], dat[b][2], sgg[b]).wait()

    fire_idx(0, 0)
    wait_idx(0, 0)
    fire_g(0)
    fire_idx(1, 1)

    @pl.loop(0, CH // 2)
    def _(jo):
        for b in range(2):
            j = jo * 2 + b
            nb = 1 - b
            wait_g(b)

            @pl.when(j < CH - 1)
            def _():
                wait_idx(j + 1, nb)
                fire_g(nb)

            as_v, ad_v, he_v = dat[b]

            @pl.loop(0, K)
            def _(r):
                t = _leaky(as_v[r] + ad_v[r, pl.ds(0, 16)]) - mvec
                al = jnp.exp(t) / ad_v[r, pl.ds(16, 16)]
                al_v[r] = al
                acc0 = jnp.zeros((16,), jnp.float32)
                acc1 = jnp.zeros((16,), jnp.float32)
                for hh in range(H):
                    a = al[hh]
                    acc0 = acc0 + a * he_v[r, pl.ds(hh * 32, 16)]
                    acc1 = acc1 + a * he_v[r, pl.ds(hh * 32 + 16, 16)]
                msg_v[r, pl.ds(0, 16)] = acc0
                msg_v[r, pl.ds(16, 16)] = acc1

            pltpu.sync_copy(msg_v, emb_sh.at[idx[b][1]], add=True)
            pltpu.sync_copy(al_v, w_sh.at[idx[b][0]], add=True)

            @pl.when(j < CH - 2)
            def _():
                fire_idx(j + 2, b)

    plsc.subcore_barrier()
    pltpu.sync_copy(emb_sh.at[pl.ds(sid * NP_T, NP_T)],
                    epart_hbm.at[cid, pl.ds(sid * NP_T, NP_T)])
    pltpu.sync_copy(w_sh.at[pl.ds(sid * NP_T, NP_T)],
                    wpart_hbm.at[cid, pl.ds(sid * NP_T, NP_T)])


def _sc2(src, dst, a_s, addn, he, m16):
    mesh = plsc.VectorSubcoreMesh(core_axis_name="c", subcore_axis_name="s", num_cores=NC, num_subcores=NS)
    f = pl.kernel(
        _sc2_body,
        out_type=(jax.ShapeDtypeStruct((NC, NP, 32), jnp.float32),
                  jax.ShapeDtypeStruct((NC, NP, 16), jnp.float32)),
        mesh=mesh,
        compiler_params=pltpu.CompilerParams(use_tc_tiling_on_sc=False),
        scratch_types=[
            pltpu.VMEM_SHARED((NP, 32), jnp.float32),
            pltpu.VMEM_SHARED((NP, 16), jnp.float32),
            pltpu.VMEM((K,), jnp.int32),
            pltpu.VMEM((K,), jnp.int32),
            pltpu.VMEM((K,), jnp.int32),
            pltpu.VMEM((K,), jnp.int32),
            pltpu.VMEM((K, 16), jnp.float32),
            pltpu.VMEM((K, 32), jnp.float32),
            pltpu.VMEM((K, HE), jnp.float32),
            pltpu.VMEM((K, 16), jnp.float32),
            pltpu.VMEM((K, 32), jnp.float32),
            pltpu.VMEM((K, HE), jnp.float32),
            pltpu.VMEM((K, 16), jnp.float32),
            pltpu.VMEM((K, 32), jnp.float32),
            pltpu.VMEM((ZCH, 32), jnp.float32),
            pltpu.VMEM((ZCH, 16), jnp.float32),
            pltpu.VMEM((16,), jnp.float32),
            pltpu.SemaphoreType.DMA,
            pltpu.SemaphoreType.DMA,
            pltpu.SemaphoreType.DMA,
            pltpu.SemaphoreType.DMA,
        ],
    )
    return f(src, dst, a_s, addn, he, m16)


def _tc_b1_body(h_ref, wp_ref, p16_ref, wpool_ref, bg_ref, bp_ref,
                g_ref, acc_ref):
    i = pl.program_id(0)
    w_blk = wp_ref[0] + wp_ref[1]
    wexp = jnp.dot(w_blk, p16_ref[...],
                   preferred_element_type=jnp.float32)
    part = jnp.sum(h_ref[...] * wexp, axis=0, keepdims=True)

    @pl.when(i == 0)
    def _():
        acc_ref[...] = part

    @pl.when(i > 0)
    def _():
        acc_ref[...] = acc_ref[...] + part

    @pl.when(i == pl.num_programs(0) - 1)
    def _():
        m = acc_ref[...] / jnp.float32(N) + bg_ref[...]
        g_ref[...] = jnp.maximum(
            jnp.dot(m, wpool_ref[...], preferred_element_type=jnp.float32)
            + bp_ref[...], 0.0)


def _tc_b1(h, wpart, p16, w_pool, b_gat, b_pool):
    grid = N // BLK
    return pl.pallas_call(
        _tc_b1_body,
        grid=(grid,),
        in_specs=[
            pl.BlockSpec((BLK, HC), lambda i: (i, 0)),
            pl.BlockSpec((NC, BLK, 16), lambda i: (0, i, 0)),
            pl.BlockSpec((16, HC), lambda i: (0, 0)),
            pl.BlockSpec((HC, D_HID), lambda i: (0, 0)),
            pl.BlockSpec((1, HC), lambda i: (0, 0)),
            pl.BlockSpec((1, D_HID), lambda i: (0, 0)),
        ],
        out_specs=pl.BlockSpec((1, D_HID), lambda i: (0, 0)),
        out_shape=jax.ShapeDtypeStruct((1, D_HID), jnp.float32),
        scratch_shapes=[pltpu.VMEM((1, HC), jnp.float32)],
    )(h, wpart, p16, w_pool, b_gat, b_pool)


def _tc_b2_body(ep_ref, ebias_ref, g_ref, wi1_ref, bi1_ref, wi2_ref, bi2_ref,
                emb_ref, inf_ref):
    emb = ep_ref[0] + ep_ref[1] + ebias_ref[...]
    emb_ref[...] = emb
    gb = jnp.broadcast_to(g_ref[...], (BLK, D_HID))
    comb = jnp.concatenate([emb, gb], axis=1)
    hid = jnp.maximum(
        jnp.dot(comb, wi1_ref[...], preferred_element_type=jnp.float32)
        + bi1_ref[...], 0.0)
    z = jnp.dot(hid, wi2_ref[...], preferred_element_type=jnp.float32) \
        + bi2_ref[...]
    inf_ref[...] = 1.0 / (1.0 + jnp.exp(-z))


def _tc_b2(epart, ebias, g, w_i1, b_i1, w_i2, b_i2):
    grid = N // BLK
    return pl.pallas_call(
        _tc_b2_body,
        grid=(grid,),
        in_specs=[
            pl.BlockSpec((NC, BLK, D_EMB), lambda i: (0, i, 0)),
            pl.BlockSpec((1, D_EMB), lambda i: (0, 0)),
            pl.BlockSpec((1, D_HID), lambda i: (0, 0)),
            pl.BlockSpec((D_EMB + D_HID, 64), lambda i: (0, 0)),
            pl.BlockSpec((1, 64), lambda i: (0, 0)),
            pl.BlockSpec((64, 1), lambda i: (0, 0)),
            pl.BlockSpec((1, 1), lambda i: (0, 0)),
        ],
        out_specs=[
            pl.BlockSpec((BLK, D_EMB), lambda i: (i, 0)),
            pl.BlockSpec((BLK, 1), lambda i: (i, 0)),
        ],
        out_shape=[
            jax.ShapeDtypeStruct((N, D_EMB), jnp.float32),
            jax.ShapeDtypeStruct((N, 1), jnp.float32),
        ],
    )(epart, ebias, g, w_i1, b_i1, w_i2, b_i2)


def kernel(x, edge_index, W_gat, att_src, att_dst, b_gat, W_pool, b_pool,
           W_emb, b_emb, W_i1, b_i1, W_i2, b_i2):
    asrc_flat = att_src.reshape(1, HC)
    adst_flat = att_dst.reshape(1, HC)
    bd = jnp.zeros((H, C, H, D_EMB), jnp.float32)
    ii = jnp.arange(H)
    bd = bd.at[ii, :, ii, :].set(W_emb.reshape(H, C, D_EMB))
    bd = bd.reshape(HC, HE)
    p16 = jnp.zeros((16, H, C), jnp.float32)
    p16 = p16.at[jnp.arange(H), jnp.arange(H), :].set(1.0)
    p16 = p16.reshape(16, HC)

    h, he, a_s, a_d, mx = _tc_a(x, W_gat, bd, asrc_flat, adst_flat)
    m16 = _leaky(mx[0] + mx[1])

    loop_idx = jnp.arange(N, dtype=jnp.int32)
    src = jnp.concatenate([edge_index[0].astype(jnp.int32), loop_idx,
                           jnp.full((E_PAD - E_TOT,), N, jnp.int32)])
    dst = jnp.concatenate([edge_index[1].astype(jnp.int32), loop_idx,
                           jnp.full((E_PAD - E_TOT,), N, jnp.int32)])

    pad_rows = NP - N
    a_s_p = jnp.concatenate([a_s, jnp.zeros((pad_rows, 16), jnp.float32)])
    a_d_p = jnp.concatenate([a_d, jnp.zeros((pad_rows, 16), jnp.float32)])
    he_p = jnp.concatenate([he, jnp.zeros((pad_rows, HE), jnp.float32)])

    dpart = _sc1(src, dst, a_s_p, a_d_p, m16)
    denom = dpart[0] + dpart[1]

    addn = jnp.concatenate([a_d_p, denom], axis=1)
    epart, wpart = _sc2(src, dst, a_s_p, addn, he_p, m16)

    g = _tc_b1(h, wpart[:, :N, :], p16, W_pool, b_gat.reshape(1, HC),
               b_pool.reshape(1, D_HID))
    ebias = (b_gat @ W_emb + b_emb).reshape(1, D_EMB)
    emb, inf = _tc_b2(epart[:, :N, :], ebias, g, W_i1,
                      b_i1.reshape(1, 64), W_i2, b_i2.reshape(1, 1))
    return (emb, inf)


# speedup vs baseline: 62.4145x; 1.2644x over previous
import functools

import jax
import jax.numpy as jnp
from jax import lax
from jax.experimental import pallas as pl
from jax.experimental.pallas import tpu as pltpu
from jax.experimental.pallas import tpu_sc as plsc

N = 10000
E = 320000
D_IN = 128
H = 8
C = 128
HC = H * C
D_EMB = 32
HE = H * D_EMB
D_HID = 128

NC = 2
NS = 16
NW = NC * NS
K = 128
E_TOT = E + N
CH = 2 * (-(-E_TOT // (NW * K * 2)))
E_PAD = NW * K * CH
NP = 10112
NP_T = NP // NS

BLK = 1000


def _tc_a_body(x_ref, wg_ref, bd_ref, asrc_ref, adst_ref,
               h_ref, he_ref, as_ref, ad_ref, mx_ref):
    i = pl.program_id(0)
    xb = x_ref[...]
    hb = jnp.dot(xb, wg_ref[...], preferred_element_type=jnp.float32)
    h_ref[...] = hb
    he_ref[...] = jnp.dot(
        hb, bd_ref[...],
        preferred_element_type=jnp.float32).astype(jnp.bfloat16)
    zs = jnp.zeros((BLK, 8), dtype=jnp.float32)
    ts = hb * asrc_ref[...]
    td = hb * adst_ref[...]
    a_s = jnp.concatenate(
        [jnp.sum(ts[:, hh * C:(hh + 1) * C], axis=1, keepdims=True)
         for hh in range(H)] + [zs], axis=1)
    a_d = jnp.concatenate(
        [jnp.sum(td[:, hh * C:(hh + 1) * C], axis=1, keepdims=True)
         for hh in range(H)] + [zs], axis=1)
    as_ref[...] = a_s
    ad_ref[...] = a_d
    ms = jnp.max(a_s, axis=0, keepdims=True)
    md = jnp.max(a_d, axis=0, keepdims=True)
    upd = jnp.concatenate([ms, md, jnp.full((6, 16), -jnp.inf, jnp.float32)],
                          axis=0)

    @pl.when(i == 0)
    def _():
        mx_ref[...] = upd

    @pl.when(i > 0)
    def _():
        mx_ref[...] = jnp.maximum(mx_ref[...], upd)


def _tc_a(x, w_gat, bd, asrc_flat, adst_flat):
    grid = N // BLK
    return pl.pallas_call(
        _tc_a_body,
        grid=(grid,),
        in_specs=[
            pl.BlockSpec((BLK, D_IN), lambda i: (i, 0)),
            pl.BlockSpec((D_IN, HC), lambda i: (0, 0)),
            pl.BlockSpec((HC, HE), lambda i: (0, 0)),
            pl.BlockSpec((1, HC), lambda i: (0, 0)),
            pl.BlockSpec((1, HC), lambda i: (0, 0)),
        ],
        out_specs=[
            pl.BlockSpec((BLK, HC), lambda i: (i, 0)),
            pl.BlockSpec((BLK, HE), lambda i: (i, 0)),
            pl.BlockSpec((BLK, 16), lambda i: (i, 0)),
            pl.BlockSpec((BLK, 16), lambda i: (i, 0)),
            pl.BlockSpec((8, 16), lambda i: (0, 0)),
        ],
        out_shape=[
            jax.ShapeDtypeStruct((N, HC), jnp.float32),
            jax.ShapeDtypeStruct((N, HE), jnp.bfloat16),
            jax.ShapeDtypeStruct((N, 16), jnp.float32),
            jax.ShapeDtypeStruct((N, 16), jnp.float32),
            jax.ShapeDtypeStruct((8, 16), jnp.float32),
        ],
    )(x, w_gat, bd, asrc_flat, adst_flat)


def _leaky(t):
    return jnp.where(t >= 0.0, t, 0.2 * t)


ZCH = NP_T // 4


def _sc1_body(src_hbm, dst_hbm, as_hbm, ad_hbm, m_hbm, dpart_hbm,
              den_sh, src0, dst0, src1, dst1, as0, ad0, as1, ad1,
              s_v, z_v, m_v, semi0, semi1, semg0, semg1):
    cid = lax.axis_index("c")
    sid = lax.axis_index("s")
    wid = cid * NS + sid

    @pl.loop(0, ZCH)
    def _(r):
        z_v[r] = jnp.zeros((16,), jnp.float32)
    for q in range(4):
        pltpu.sync_copy(z_v, den_sh.at[pl.ds(sid * NP_T + q * ZCH, ZCH)])
    pltpu.sync_copy(m_hbm, m_v)
    plsc.subcore_barrier()

    mvec = m_v[...]
    tb = wid * (CH * K)
    idx = ((src0, dst0), (src1, dst1))
    dat = ((as0, ad0), (as1, ad1))
    sgi = (semi0, semi1)
    sgg = (semg0, semg1)

    def fire_idx(j, b):
        pltpu.async_copy(src_hbm.at[pl.ds(tb + j * K, K)], idx[b][0], sgi[b])
        pltpu.async_copy(dst_hbm.at[pl.ds(tb + j * K, K)], idx[b][1], sgi[b])

    def wait_idx(j, b):
        pltpu.make_async_copy(src_hbm.at[pl.ds(tb + j * K, K)], idx[b][0],
                              sgi[b]).wait()
        pltpu.make_async_copy(dst_hbm.at[pl.ds(tb + j * K, K)], idx[b][1],
                              sgi[b]).wait()

    def fire_g(b):
        pltpu.async_copy(as_hbm.at[idx[b][0]], dat[b][0], sgg[b])
        pltpu.async_copy(ad_hbm.at[idx[b][1]], dat[b][1], sgg[b])

    def wait_g(b):
        pltpu.make_async_copy(as_hbm.at[idx[b][0]], dat[b][0], sgg[b]).wait()
        pltpu.make_async_copy(ad_hbm.at[idx[b][1]], dat[b][1], sgg[b]).wait()

    fire_idx(0, 0)
    wait_idx(0, 0)
    fire_g(0)
    fire_idx(1, 1)

    @pl.loop(0, CH // 2)
    def _(jo):
        for b in range(2):
            j = jo * 2 + b
            nb = 1 - b
            wait_g(b)

            @pl.when(j < CH - 1)
            def _():
                wait_idx(j + 1, nb)
                fire_g(nb)

            as_v, ad_v = dat[b]

            @pl.loop(0, K)
            def _(r):
                t = _leaky(as_v[r] + ad_v[r]) - mvec
                s_v[r] = jnp.exp(t)

            pltpu.sync_copy(s_v, den_sh.at[idx[b][1]], add=True)

            @pl.when(j < CH - 2)
            def _():
                fire_idx(j + 2, b)

    plsc.subcore_barrier()
    pltpu.sync_copy(den_sh.at[pl.ds(sid * NP_T, NP_T)],
                    dpart_hbm.at[cid, pl.ds(sid * NP_T, NP_T)])


def _sc1(src, dst, a_s, a_d, m16):
    mesh = plsc.VectorSubcoreMesh(core_axis_name="c", subcore_axis_name="s", num_cores=NC, num_subcores=NS)
    f = pl.kernel(
        _sc1_body,
        out_type=jax.ShapeDtypeStruct((NC, NP, 16), jnp.float32),
        mesh=mesh,
        compiler_params=pltpu.CompilerParams(use_tc_tiling_on_sc=False),
        scratch_types=[
            pltpu.VMEM_SHARED((NP, 16), jnp.float32),
            pltpu.VMEM((K,), jnp.int32),
            pltpu.VMEM((K,), jnp.int32),
            pltpu.VMEM((K,), jnp.int32),
            pltpu.VMEM((K,), jnp.int32),
            pltpu.VMEM((K, 16), jnp.float32),
            pltpu.VMEM((K, 16), jnp.float32),
            pltpu.VMEM((K, 16), jnp.float32),
            pltpu.VMEM((K, 16), jnp.float32),
            pltpu.VMEM((K, 16), jnp.float32),
            pltpu.VMEM((ZCH, 16), jnp.float32),
            pltpu.VMEM((16,), jnp.float32),
            pltpu.SemaphoreType.DMA,
            pltpu.SemaphoreType.DMA,
            pltpu.SemaphoreType.DMA,
            pltpu.SemaphoreType.DMA,
        ],
    )
    return f(src, dst, a_s, a_d, m16)


def _sc2_body(src_hbm, dst_hbm, as_hbm, addn_hbm, he_hbm, m_hbm,
              epart_hbm, wpart_hbm,
              emb_sh, w_sh, src0, dst0, src1, dst1,
              as0, ad0, he0, as1, ad1, he1,
              al_v, msg_v, z_v, z16_v, m_v, semi0, semi1, semg0, semg1):
    cid = lax.axis_index("c")
    sid = lax.axis_index("s")
    wid = cid * NS + sid

    @pl.loop(0, ZCH)
    def _(r):
        z_v[r] = jnp.zeros((32,), jnp.float32)
        z16_v[r] = jnp.zeros((16,), jnp.float32)
    for q in range(4):
        pltpu.sync_copy(z_v, emb_sh.at[pl.ds(sid * NP_T + q * ZCH, ZCH)])
        pltpu.sync_copy(z16_v, w_sh.at[pl.ds(sid * NP_T + q * ZCH, ZCH)])
    pltpu.sync_copy(m_hbm, m_v)
    plsc.subcore_barrier()

    mvec = m_v[...]
    tb = wid * (CH * K)
    idx = ((src0, dst0), (src1, dst1))
    dat = ((as0, ad0, he0), (as1, ad1, he1))
    sgi = (semi0, semi1)
    sgg = (semg0, semg1)

    def fire_idx(j, b):
        pltpu.async_copy(src_hbm.at[pl.ds(tb + j * K, K)], idx[b][0], sgi[b])
        pltpu.async_copy(dst_hbm.at[pl.ds(tb + j * K, K)], idx[b][1], sgi[b])

    def wait_idx(j, b):
        pltpu.make_async_copy(src_hbm.at[pl.ds(tb + j * K, K)], idx[b][0],
                              sgi[b]).wait()
        pltpu.make_async_copy(dst_hbm.at[pl.ds(tb + j * K, K)], idx[b][1],
                              sgi[b]).wait()

    def fire_g(b):
        pltpu.async_copy(as_hbm.at[idx[b][0]], dat[b][0], sgg[b])
        pltpu.async_copy(addn_hbm.at[idx[b][1]], dat[b][1], sgg[b])
        pltpu.async_copy(he_hbm.at[idx[b][0]], dat[b][2], sgg[b])

    def wait_g(b):
        pltpu.make_async_copy(as_hbm.at[idx[b][0]], dat[b][0], sgg[b]).wait()
        pltpu.make_async_copy(addn_hbm.at[idx[b][1]], dat[b][1],
                              sgg[b]).wait()
        pltpu.make_async_copy(he_hbm.at[idx[b][0]], dat[b][2], sgg[b]).wait()

    fire_idx(0, 0)
    wait_idx(0, 0)
    fire_g(0)
    fire_idx(1, 1)

    @pl.loop(0, CH // 2)
    def _(jo):
        for b in range(2):
            j = jo * 2 + b
            nb = 1 - b
            wait_g(b)

            @pl.when(j < CH - 1)
            def _():
                wait_idx(j + 1, nb)
                fire_g(nb)

            as_v, ad_v, he_v = dat[b]

            @pl.loop(0, K)
            def _(r):
                t = _leaky(as_v[r] + ad_v[r, pl.ds(0, 16)]) - mvec
                al = jnp.exp(t) / ad_v[r, pl.ds(16, 16)]
                al_v[r] = al
                acc0 = jnp.zeros((16,), jnp.float32)
                acc1 = jnp.zeros((16,), jnp.float32)
                for hh in range(H):
                    a = al[hh]
                    u = he_v[r, pl.ds(hh * 16, 16)]
                    va = lax.bitcast_convert_type(u << 16, jnp.float32)
                    vb = lax.bitcast_convert_type(u & jnp.int32(-65536),
                                                  jnp.float32)
                    acc0 = acc0 + a * va
                    acc1 = acc1 + a * vb
                msg_v[r, pl.ds(0, 16)] = acc0
                msg_v[r, pl.ds(16, 16)] = acc1

            pltpu.sync_copy(msg_v, emb_sh.at[idx[b][1]], add=True)
            pltpu.sync_copy(al_v, w_sh.at[idx[b][0]], add=True)

            @pl.when(j < CH - 2)
            def _():
                fire_idx(j + 2, b)

    plsc.subcore_barrier()
    pltpu.sync_copy(emb_sh.at[pl.ds(sid * NP_T, NP_T)],
                    epart_hbm.at[cid, pl.ds(sid * NP_T, NP_T)])
    pltpu.sync_copy(w_sh.at[pl.ds(sid * NP_T, NP_T)],
                    wpart_hbm.at[cid, pl.ds(sid * NP_T, NP_T)])


def _sc2(src, dst, a_s, addn, he, m16):
    mesh = plsc.VectorSubcoreMesh(core_axis_name="c", subcore_axis_name="s", num_cores=NC, num_subcores=NS)
    f = pl.kernel(
        _sc2_body,
        out_type=(jax.ShapeDtypeStruct((NC, NP, 32), jnp.float32),
                  jax.ShapeDtypeStruct((NC, NP, 16), jnp.float32)),
        mesh=mesh,
        compiler_params=pltpu.CompilerParams(use_tc_tiling_on_sc=False),
        scratch_types=[
            pltpu.VMEM_SHARED((NP, 32), jnp.float32),
            pltpu.VMEM_SHARED((NP, 16), jnp.float32),
            pltpu.VMEM((K,), jnp.int32),
            pltpu.VMEM((K,), jnp.int32),
            pltpu.VMEM((K,), jnp.int32),
            pltpu.VMEM((K,), jnp.int32),
            pltpu.VMEM((K, 16), jnp.float32),
            pltpu.VMEM((K, 32), jnp.float32),
            pltpu.VMEM((K, HE // 2), jnp.int32),
            pltpu.VMEM((K, 16), jnp.float32),
            pltpu.VMEM((K, 32), jnp.float32),
            pltpu.VMEM((K, HE // 2), jnp.int32),
            pltpu.VMEM((K, 16), jnp.float32),
            pltpu.VMEM((K, 32), jnp.float32),
            pltpu.VMEM((ZCH, 32), jnp.float32),
            pltpu.VMEM((ZCH, 16), jnp.float32),
            pltpu.VMEM((16,), jnp.float32),
            pltpu.SemaphoreType.DMA,
            pltpu.SemaphoreType.DMA,
            pltpu.SemaphoreType.DMA,
            pltpu.SemaphoreType.DMA,
        ],
    )
    return f(src, dst, a_s, addn, he, m16)


def _tc_b1_body(h_ref, wp_ref, p16_ref, wpool_ref, bg_ref, bp_ref,
                g_ref, acc_ref):
    i = pl.program_id(0)
    w_blk = wp_ref[0] + wp_ref[1]
    wexp = jnp.dot(w_blk, p16_ref[...],
                   preferred_element_type=jnp.float32)
    part = jnp.sum(h_ref[...] * wexp, axis=0, keepdims=True)

    @pl.when(i == 0)
    def _():
        acc_ref[...] = part

    @pl.when(i > 0)
    def _():
        acc_ref[...] = acc_ref[...] + part

    @pl.when(i == pl.num_programs(0) - 1)
    def _():
        m = acc_ref[...] / jnp.float32(N) + bg_ref[...]
        g_ref[...] = jnp.maximum(
            jnp.dot(m, wpool_ref[...], preferred_element_type=jnp.float32)
            + bp_ref[...], 0.0)


def _tc_b1(h, wpart, p16, w_pool, b_gat, b_pool):
    grid = N // BLK
    return pl.pallas_call(
        _tc_b1_body,
        grid=(grid,),
        in_specs=[
            pl.BlockSpec((BLK, HC), lambda i: (i, 0)),
            pl.BlockSpec((NC, BLK, 16), lambda i: (0, i, 0)),
            pl.BlockSpec((16, HC), lambda i: (0, 0)),
            pl.BlockSpec((HC, D_HID), lambda i: (0, 0)),
            pl.BlockSpec((1, HC), lambda i: (0, 0)),
            pl.BlockSpec((1, D_HID), lambda i: (0, 0)),
        ],
        out_specs=pl.BlockSpec((1, D_HID), lambda i: (0, 0)),
        out_shape=jax.ShapeDtypeStruct((1, D_HID), jnp.float32),
        scratch_shapes=[pltpu.VMEM((1, HC), jnp.float32)],
    )(h, wpart, p16, w_pool, b_gat, b_pool)


def _tc_b2_body(ep_ref, ebias_ref, g_ref, wi1_ref, bi1_ref, wi2_ref, bi2_ref,
                emb_ref, inf_ref):
    emb = ep_ref[0] + ep_ref[1] + ebias_ref[...]
    emb_ref[...] = emb
    gb = jnp.broadcast_to(g_ref[...], (BLK, D_HID))
    comb = jnp.concatenate([emb, gb], axis=1)
    hid = jnp.maximum(
        jnp.dot(comb, wi1_ref[...], preferred_element_type=jnp.float32)
        + bi1_ref[...], 0.0)
    z = jnp.dot(hid, wi2_ref[...], preferred_element_type=jnp.float32) \
        + bi2_ref[...]
    inf_ref[...] = 1.0 / (1.0 + jnp.exp(-z))


def _tc_b2(epart, ebias, g, w_i1, b_i1, w_i2, b_i2):
    grid = N // BLK
    return pl.pallas_call(
        _tc_b2_body,
        grid=(grid,),
        in_specs=[
            pl.BlockSpec((NC, BLK, D_EMB), lambda i: (0, i, 0)),
            pl.BlockSpec((1, D_EMB), lambda i: (0, 0)),
            pl.BlockSpec((1, D_HID), lambda i: (0, 0)),
            pl.BlockSpec((D_EMB + D_HID, 64), lambda i: (0, 0)),
            pl.BlockSpec((1, 64), lambda i: (0, 0)),
            pl.BlockSpec((64, 1), lambda i: (0, 0)),
            pl.BlockSpec((1, 1), lambda i: (0, 0)),
        ],
        out_specs=[
            pl.BlockSpec((BLK, D_EMB), lambda i: (i, 0)),
            pl.BlockSpec((BLK, 1), lambda i: (i, 0)),
        ],
        out_shape=[
            jax.ShapeDtypeStruct((N, D_EMB), jnp.float32),
            jax.ShapeDtypeStruct((N, 1), jnp.float32),
        ],
    )(epart, ebias, g, w_i1, b_i1, w_i2, b_i2)


def kernel(x, edge_index, W_gat, att_src, att_dst, b_gat, W_pool, b_pool,
           W_emb, b_emb, W_i1, b_i1, W_i2, b_i2):
    asrc_flat = att_src.reshape(1, HC)
    adst_flat = att_dst.reshape(1, HC)
    bd = jnp.zeros((H, C, H, D_EMB), jnp.float32)
    ii = jnp.arange(H)
    bd = bd.at[ii, :, ii, :].set(W_emb.reshape(H, C, D_EMB))
    bd = bd.reshape(HC, HE)
    blk_perm = jnp.stack(
        [jnp.arange(16), jnp.arange(16, 32)], axis=1).reshape(32)
    perm_cols = (jnp.arange(H)[:, None] * 32 + blk_perm[None, :]).reshape(HE)
    bd = bd[:, perm_cols]
    p16 = jnp.zeros((16, H, C), jnp.float32)
    p16 = p16.at[jnp.arange(H), jnp.arange(H), :].set(1.0)
    p16 = p16.reshape(16, HC)

    h, he, a_s, a_d, mx = _tc_a(x, W_gat, bd, asrc_flat, adst_flat)
    m16 = _leaky(mx[0] + mx[1])

    loop_idx = jnp.arange(N, dtype=jnp.int32)
    pad_idx = N + jnp.arange(E_PAD - E_TOT, dtype=jnp.int32) % (NP - N)
    src = jnp.concatenate([edge_index[0].astype(jnp.int32), loop_idx,
                           pad_idx])
    dst = jnp.concatenate([edge_index[1].astype(jnp.int32), loop_idx,
                           pad_idx])

    pad_rows = NP - N
    a_s_p = jnp.concatenate([a_s, jnp.zeros((pad_rows, 16), jnp.float32)])
    a_d_p = jnp.concatenate([a_d, jnp.zeros((pad_rows, 16), jnp.float32)])
    he_p = jnp.concatenate([he, jnp.zeros((pad_rows, HE), jnp.bfloat16)])
    he_p = lax.bitcast_convert_type(
        he_p.reshape(NP, HE // 2, 2), jnp.int32)

    dpart = _sc1(src, dst, a_s_p, a_d_p, m16)
    denom = dpart[0] + dpart[1]

    addn = jnp.concatenate([a_d_p, denom], axis=1)
    epart, wpart = _sc2(src, dst, a_s_p, addn, he_p, m16)

    g = _tc_b1(h, wpart[:, :N, :], p16, W_pool, b_gat.reshape(1, HC),
               b_pool.reshape(1, D_HID))
    ebias = (b_gat @ W_emb + b_emb).reshape(1, D_EMB)
    emb, inf = _tc_b2(epart[:, :N, :], ebias, g, W_i1,
                      b_i1.reshape(1, 64), W_i2, b_i2.reshape(1, 1))
    return (emb, inf)


# speedup vs baseline: 65.7564x; 1.0535x over previous
import functools

import jax
import jax.numpy as jnp
from jax import lax
from jax.experimental import pallas as pl
from jax.experimental.pallas import tpu as pltpu
from jax.experimental.pallas import tpu_sc as plsc

N = 10000
E = 320000
D_IN = 128
H = 8
C = 128
HC = H * C
D_EMB = 32
HE = H * D_EMB
D_HID = 128

NC = 2
NS = 16
NW = NC * NS
K = 128
E_TOT = E + N
CH = 2 * (-(-E_TOT // (NW * K * 2)))
E_PAD = NW * K * CH
NP = 10112
NP_T = NP // NS

BLK = 1000


def _tc_a_body(x_ref, wg_ref, bd_ref, asrc_ref, adst_ref,
               h_ref, he_ref, as_ref, ad_ref, mx_ref):
    i = pl.program_id(0)
    xb = x_ref[...]
    hb = jnp.dot(xb, wg_ref[...], preferred_element_type=jnp.float32)
    h_ref[...] = hb
    he_ref[...] = jnp.dot(
        hb, bd_ref[...],
        preferred_element_type=jnp.float32).astype(jnp.bfloat16)
    zs = jnp.zeros((BLK, 8), dtype=jnp.float32)
    ts = hb * asrc_ref[...]
    td = hb * adst_ref[...]
    a_s = jnp.concatenate(
        [jnp.sum(ts[:, hh * C:(hh + 1) * C], axis=1, keepdims=True)
         for hh in range(H)] + [zs], axis=1)
    a_d = jnp.concatenate(
        [jnp.sum(td[:, hh * C:(hh + 1) * C], axis=1, keepdims=True)
         for hh in range(H)] + [zs], axis=1)
    as_ref[...] = a_s
    ad_ref[...] = a_d
    ms = jnp.max(a_s, axis=0, keepdims=True)
    md = jnp.max(a_d, axis=0, keepdims=True)
    upd = jnp.concatenate([ms, md, jnp.full((6, 16), -jnp.inf, jnp.float32)],
                          axis=0)

    @pl.when(i == 0)
    def _():
        mx_ref[...] = upd

    @pl.when(i > 0)
    def _():
        mx_ref[...] = jnp.maximum(mx_ref[...], upd)


def _tc_a(x, w_gat, bd, asrc_flat, adst_flat):
    grid = N // BLK
    return pl.pallas_call(
        _tc_a_body,
        grid=(grid,),
        in_specs=[
            pl.BlockSpec((BLK, D_IN), lambda i: (i, 0)),
            pl.BlockSpec((D_IN, HC), lambda i: (0, 0)),
            pl.BlockSpec((HC, HE), lambda i: (0, 0)),
            pl.BlockSpec((1, HC), lambda i: (0, 0)),
            pl.BlockSpec((1, HC), lambda i: (0, 0)),
        ],
        out_specs=[
            pl.BlockSpec((BLK, HC), lambda i: (i, 0)),
            pl.BlockSpec((BLK, HE), lambda i: (i, 0)),
            pl.BlockSpec((BLK, 16), lambda i: (i, 0)),
            pl.BlockSpec((BLK, 16), lambda i: (i, 0)),
            pl.BlockSpec((8, 16), lambda i: (0, 0)),
        ],
        out_shape=[
            jax.ShapeDtypeStruct((N, HC), jnp.float32),
            jax.ShapeDtypeStruct((N, HE), jnp.bfloat16),
            jax.ShapeDtypeStruct((N, 16), jnp.float32),
            jax.ShapeDtypeStruct((N, 16), jnp.float32),
            jax.ShapeDtypeStruct((8, 16), jnp.float32),
        ],
    )(x, w_gat, bd, asrc_flat, adst_flat)


def _leaky(t):
    return jnp.where(t >= 0.0, t, 0.2 * t)


ZCH = NP_T // 4


def _sc1_body(ei_hbm, as_hbm, ad_hbm, m_hbm, dpart_hbm,
              den_sh, i0, i1, i2, i3, as0, ad0, as1, ad1,
              s0, s1, z_v, m_v, semi, semg0, semg1, sems0, sems1):
    cid = lax.axis_index("c")
    sid = lax.axis_index("s")
    wid = cid * NS + sid

    @pl.loop(0, ZCH)
    def _(r):
        z_v[r] = jnp.zeros((16,), jnp.float32)
    for q in range(4):
        pltpu.sync_copy(z_v, den_sh.at[pl.ds(sid * NP_T + q * ZCH, ZCH)])
    pltpu.sync_copy(m_hbm, m_v)
    plsc.subcore_barrier()

    mvec = m_v[...]
    cb = wid * CH
    islot = (i0, i1, i2, i3)
    dat = ((as0, ad0), (as1, ad1))
    sv = (s0, s1)
    sgg = (semg0, semg1)
    sgs = (sems0, sems1)

    def fire_idx(j, q):
        pltpu.async_copy(ei_hbm.at[cb + j], islot[q], semi)

    def wait_idx(j, q):
        pltpu.make_async_copy(ei_hbm.at[cb + j], islot[q], semi).wait()

    def fire_g(q, b):
        pltpu.async_copy(as_hbm.at[islot[q].at[0]], dat[b][0], sgg[b])
        pltpu.async_copy(ad_hbm.at[islot[q].at[1]], dat[b][1], sgg[b])

    def wait_g(q, b):
        pltpu.make_async_copy(as_hbm.at[islot[q].at[0]], dat[b][0],
                              sgg[b]).wait()
        pltpu.make_async_copy(ad_hbm.at[islot[q].at[1]], dat[b][1],
                              sgg[b]).wait()

    def drain_scat(q, b):
        pltpu.make_async_copy(sv[b], den_sh.at[islot[q].at[1]],
                              sgs[b]).wait()

    fire_idx(0, 0)
    wait_idx(0, 0)
    fire_g(0, 0)
    fire_idx(1, 1)

    @pl.loop(0, CH // 2)
    def _(jo):
        for b in range(2):
            j = jo * 2 + b

            def body(q, qn, qp):
                wait_g(q, b)

                @pl.when(j >= 2)
                def _():
                    drain_scat(qp, b)

                @pl.when(j < CH - 1)
                def _():
                    wait_idx(j + 1, qn)
                    fire_g(qn, 1 - b)

                as_v, ad_v = dat[b]
                s_v = sv[b]

                @pl.loop(0, K)
                def _(r):
                    t = _leaky(as_v[r] + ad_v[r]) - mvec
                    s_v[r] = jnp.exp(t)

                pltpu.async_copy(s_v, den_sh.at[islot[q].at[1]], sgs[b],
                                 add=True)

                @pl.when(j < CH - 2)
                def _():
                    fire_idx(j + 2, qp)

            @pl.when(jo % 2 == 0)
            def _():
                body(b, b + 1 if b == 0 else 2, b + 2)

            @pl.when(jo % 2 == 1)
            def _():
                body(b + 2, 3 if b == 0 else 0, b)

    if CH % 4 == 0:
        drain_scat(2, 0)
        drain_scat(3, 1)
    else:
        drain_scat(0, 0)
        drain_scat(1, 1)

    plsc.subcore_barrier()
    pltpu.sync_copy(den_sh.at[pl.ds(sid * NP_T, NP_T)],
                    dpart_hbm.at[cid, pl.ds(sid * NP_T, NP_T)])


def _sc1(ei, a_s, a_d, m16):
    mesh = plsc.VectorSubcoreMesh(core_axis_name="c", subcore_axis_name="s", num_cores=NC, num_subcores=NS)
    f = pl.kernel(
        _sc1_body,
        out_type=jax.ShapeDtypeStruct((NC, NP, 16), jnp.float32),
        mesh=mesh,
        compiler_params=pltpu.CompilerParams(use_tc_tiling_on_sc=False),
        scratch_types=[
            pltpu.VMEM_SHARED((NP, 16), jnp.float32),
            pltpu.VMEM((2, K), jnp.int32),
            pltpu.VMEM((2, K), jnp.int32),
            pltpu.VMEM((2, K), jnp.int32),
            pltpu.VMEM((2, K), jnp.int32),
            pltpu.VMEM((K, 16), jnp.float32),
            pltpu.VMEM((K, 16), jnp.float32),
            pltpu.VMEM((K, 16), jnp.float32),
            pltpu.VMEM((K, 16), jnp.float32),
            pltpu.VMEM((K, 16), jnp.float32),
            pltpu.VMEM((K, 16), jnp.float32),
            pltpu.VMEM((ZCH, 16), jnp.float32),
            pltpu.VMEM((16,), jnp.float32),
            pltpu.SemaphoreType.DMA,
            pltpu.SemaphoreType.DMA,
            pltpu.SemaphoreType.DMA,
            pltpu.SemaphoreType.DMA,
            pltpu.SemaphoreType.DMA,
        ],
    )
    return f(ei, a_s, a_d, m16)


def _sc2_body(ei_hbm, as_hbm, addn_hbm, he_hbm, m_hbm,
              epart_hbm, wpart_hbm,
              emb_sh, w_sh, i0, i1, i2, i3,
              as0, ad0, he0, as1, ad1, he1,
              al0, msg0, al1, msg1, z_v, z16_v, m_v,
              semi, semg0, semg1, sems0, sems1):
    cid = lax.axis_index("c")
    sid = lax.axis_index("s")
    wid = cid * NS + sid

    @pl.loop(0, ZCH)
    def _(r):
        z_v[r] = jnp.zeros((32,), jnp.float32)
        z16_v[r] = jnp.zeros((16,), jnp.float32)
    for q in range(4):
        pltpu.sync_copy(z_v, emb_sh.at[pl.ds(sid * NP_T + q * ZCH, ZCH)])
        pltpu.sync_copy(z16_v, w_sh.at[pl.ds(sid * NP_T + q * ZCH, ZCH)])
    pltpu.sync_copy(m_hbm, m_v)
    plsc.subcore_barrier()

    mvec = m_v[...]
    cb = wid * CH
    islot = (i0, i1, i2, i3)
    dat = ((as0, ad0, he0), (as1, ad1, he1))
    sv = ((al0, msg0), (al1, msg1))
    sgg = (semg0, semg1)
    sgs = (sems0, sems1)

    def fire_idx(j, q):
        pltpu.async_copy(ei_hbm.at[cb + j], islot[q], semi)

    def wait_idx(j, q):
        pltpu.make_async_copy(ei_hbm.at[cb + j], islot[q], semi).wait()

    def fire_g(q, b):
        pltpu.async_copy(as_hbm.at[islot[q].at[0]], dat[b][0], sgg[b])
        pltpu.async_copy(addn_hbm.at[islot[q].at[1]], dat[b][1], sgg[b])
        pltpu.async_copy(he_hbm.at[islot[q].at[0]], dat[b][2], sgg[b])

    def wait_g(q, b):
        pltpu.make_async_copy(as_hbm.at[islot[q].at[0]], dat[b][0],
                              sgg[b]).wait()
        pltpu.make_async_copy(addn_hbm.at[islot[q].at[1]], dat[b][1],
                              sgg[b]).wait()
        pltpu.make_async_copy(he_hbm.at[islot[q].at[0]], dat[b][2],
                              sgg[b]).wait()

    def drain_scat(q, b):
        pltpu.make_async_copy(sv[b][1], emb_sh.at[islot[q].at[1]],
                              sgs[b]).wait()
        pltpu.make_async_copy(sv[b][0], w_sh.at[islot[q].at[0]],
                              sgs[b]).wait()

    fire_idx(0, 0)
    wait_idx(0, 0)
    fire_g(0, 0)
    fire_idx(1, 1)

    @pl.loop(0, CH // 2)
    def _(jo):
        for b in range(2):
            j = jo * 2 + b

            def body(q, qn, qp):
                wait_g(q, b)

                @pl.when(j >= 2)
                def _():
                    drain_scat(qp, b)

                @pl.when(j < CH - 1)
                def _():
                    wait_idx(j + 1, qn)
                    fire_g(qn, 1 - b)

                as_v, ad_v, he_v = dat[b]
                al_v, msg_v = sv[b]

                @pl.loop(0, K)
                def _(r):
                    t = _leaky(as_v[r] + ad_v[r, pl.ds(0, 16)]) - mvec
                    al = jnp.exp(t) / ad_v[r, pl.ds(16, 16)]
                    al_v[r] = al
                    acc0 = jnp.zeros((16,), jnp.float32)
                    acc1 = jnp.zeros((16,), jnp.float32)
                    for hh in range(H):
                        a = al[hh]
                        u = he_v[r, pl.ds(hh * 16, 16)]
                        va = lax.bitcast_convert_type(u << 16, jnp.float32)
                        vb = lax.bitcast_convert_type(
                            u & jnp.int32(-65536), jnp.float32)
                        acc0 = acc0 + a * va
                        acc1 = acc1 + a * vb
                    msg_v[r, pl.ds(0, 16)] = acc0
                    msg_v[r, pl.ds(16, 16)] = acc1

                pltpu.async_copy(msg_v, emb_sh.at[islot[q].at[1]], sgs[b],
                                 add=True)
                pltpu.async_copy(al_v, w_sh.at[islot[q].at[0]], sgs[b],
                                 add=True)

                @pl.when(j < CH - 2)
                def _():
                    fire_idx(j + 2, qp)

            @pl.when(jo % 2 == 0)
            def _():
                body(b, b + 1 if b == 0 else 2, b + 2)

            @pl.when(jo % 2 == 1)
            def _():
                body(b + 2, 3 if b == 0 else 0, b)

    if CH % 4 == 0:
        drain_scat(2, 0)
        drain_scat(3, 1)
    else:
        drain_scat(0, 0)
        drain_scat(1, 1)

    plsc.subcore_barrier()
    pltpu.sync_copy(emb_sh.at[pl.ds(sid * NP_T, NP_T)],
                    epart_hbm.at[cid, pl.ds(sid * NP_T, NP_T)])
    pltpu.sync_copy(w_sh.at[pl.ds(sid * NP_T, NP_T)],
                    wpart_hbm.at[cid, pl.ds(sid * NP_T, NP_T)])


def _sc2(ei, a_s, addn, he, m16):
    mesh = plsc.VectorSubcoreMesh(core_axis_name="c", subcore_axis_name="s", num_cores=NC, num_subcores=NS)
    f = pl.kernel(
        _sc2_body,
        out_type=(jax.ShapeDtypeStruct((NC, NP, 32), jnp.float32),
                  jax.ShapeDtypeStruct((NC, NP, 16), jnp.float32)),
        mesh=mesh,
        compiler_params=pltpu.CompilerParams(use_tc_tiling_on_sc=False),
        scratch_types=[
            pltpu.VMEM_SHARED((NP, 32), jnp.float32),
            pltpu.VMEM_SHARED((NP, 16), jnp.float32),
            pltpu.VMEM((2, K), jnp.int32),
            pltpu.VMEM((2, K), jnp.int32),
            pltpu.VMEM((2, K), jnp.int32),
            pltpu.VMEM((2, K), jnp.int32),
            pltpu.VMEM((K, 16), jnp.float32),
            pltpu.VMEM((K, 32), jnp.float32),
            pltpu.VMEM((K, HE // 2), jnp.int32),
            pltpu.VMEM((K, 16), jnp.float32),
            pltpu.VMEM((K, 32), jnp.float32),
            pltpu.VMEM((K, HE // 2), jnp.int32),
            pltpu.VMEM((K, 16), jnp.float32),
            pltpu.VMEM((K, 32), jnp.float32),
            pltpu.VMEM((K, 16), jnp.float32),
            pltpu.VMEM((K, 32), jnp.float32),
            pltpu.VMEM((ZCH, 32), jnp.float32),
            pltpu.VMEM((ZCH, 16), jnp.float32),
            pltpu.VMEM((16,), jnp.float32),
            pltpu.SemaphoreType.DMA,
            pltpu.SemaphoreType.DMA,
            pltpu.SemaphoreType.DMA,
            pltpu.SemaphoreType.DMA,
            pltpu.SemaphoreType.DMA,
        ],
    )
    return f(ei, a_s, addn, he, m16)


def _tc_b1_body(h_ref, wp_ref, p16_ref, wpool_ref, bg_ref, bp_ref,
                g_ref, acc_ref):
    i = pl.program_id(0)
    w_blk = wp_ref[0] + wp_ref[1]
    wexp = jnp.dot(w_blk, p16_ref[...],
                   preferred_element_type=jnp.float32)
    part = jnp.sum(h_ref[...] * wexp, axis=0, keepdims=True)

    @pl.when(i == 0)
    def _():
        acc_ref[...] = part

    @pl.when(i > 0)
    def _():
        acc_ref[...] = acc_ref[...] + part

    @pl.when(i == pl.num_programs(0) - 1)
    def _():
        m = acc_ref[...] / jnp.float32(N) + bg_ref[...]
        g_ref[...] = jnp.maximum(
            jnp.dot(m, wpool_ref[...], preferred_element_type=jnp.float32)
            + bp_ref[...], 0.0)


def _tc_b1(h, wpart, p16, w_pool, b_gat, b_pool):
    grid = N // BLK
    return pl.pallas_call(
        _tc_b1_body,
        grid=(grid,),
        in_specs=[
            pl.BlockSpec((BLK, HC), lambda i: (i, 0)),
            pl.BlockSpec((NC, BLK, 16), lambda i: (0, i, 0)),
            pl.BlockSpec((16, HC), lambda i: (0, 0)),
            pl.BlockSpec((HC, D_HID), lambda i: (0, 0)),
            pl.BlockSpec((1, HC), lambda i: (0, 0)),
            pl.BlockSpec((1, D_HID), lambda i: (0, 0)),
        ],
        out_specs=pl.BlockSpec((1, D_HID), lambda i: (0, 0)),
        out_shape=jax.ShapeDtypeStruct((1, D_HID), jnp.float32),
        scratch_shapes=[pltpu.VMEM((1, HC), jnp.float32)],
    )(h, wpart, p16, w_pool, b_gat, b_pool)


def _tc_b2_body(ep_ref, ebias_ref, g_ref, wi1_ref, bi1_ref, wi2_ref, bi2_ref,
                emb_ref, inf_ref):
    emb = ep_ref[0] + ep_ref[1] + ebias_ref[...]
    emb_ref[...] = emb
    gb = jnp.broadcast_to(g_ref[...], (BLK, D_HID))
    comb = jnp.concatenate([emb, gb], axis=1)
    hid = jnp.maximum(
        jnp.dot(comb, wi1_ref[...], preferred_element_type=jnp.float32)
        + bi1_ref[...], 0.0)
    z = jnp.dot(hid, wi2_ref[...], preferred_element_type=jnp.float32) \
        + bi2_ref[...]
    inf_ref[...] = 1.0 / (1.0 + jnp.exp(-z))


def _tc_b2(epart, ebias, g, w_i1, b_i1, w_i2, b_i2):
    grid = N // BLK
    return pl.pallas_call(
        _tc_b2_body,
        grid=(grid,),
        in_specs=[
            pl.BlockSpec((NC, BLK, D_EMB), lambda i: (0, i, 0)),
            pl.BlockSpec((1, D_EMB), lambda i: (0, 0)),
            pl.BlockSpec((1, D_HID), lambda i: (0, 0)),
            pl.BlockSpec((D_EMB + D_HID, 64), lambda i: (0, 0)),
            pl.BlockSpec((1, 64), lambda i: (0, 0)),
            pl.BlockSpec((64, 1), lambda i: (0, 0)),
            pl.BlockSpec((1, 1), lambda i: (0, 0)),
        ],
        out_specs=[
            pl.BlockSpec((BLK, D_EMB), lambda i: (i, 0)),
            pl.BlockSpec((BLK, 1), lambda i: (i, 0)),
        ],
        out_shape=[
            jax.ShapeDtypeStruct((N, D_EMB), jnp.float32),
            jax.ShapeDtypeStruct((N, 1), jnp.float32),
        ],
    )(epart, ebias, g, w_i1, b_i1, w_i2, b_i2)


def kernel(x, edge_index, W_gat, att_src, att_dst, b_gat, W_pool, b_pool,
           W_emb, b_emb, W_i1, b_i1, W_i2, b_i2):
    asrc_flat = att_src.reshape(1, HC)
    adst_flat = att_dst.reshape(1, HC)
    bd = jnp.zeros((H, C, H, D_EMB), jnp.float32)
    ii = jnp.arange(H)
    bd = bd.at[ii, :, ii, :].set(W_emb.reshape(H, C, D_EMB))
    bd = bd.reshape(HC, HE)
    blk_perm = jnp.stack(
        [jnp.arange(16), jnp.arange(16, 32)], axis=1).reshape(32)
    perm_cols = (jnp.arange(H)[:, None] * 32 + blk_perm[None, :]).reshape(HE)
    bd = bd[:, perm_cols]
    p16 = jnp.zeros((16, H, C), jnp.float32)
    p16 = p16.at[jnp.arange(H), jnp.arange(H), :].set(1.0)
    p16 = p16.reshape(16, HC)

    h, he, a_s, a_d, mx = _tc_a(x, W_gat, bd, asrc_flat, adst_flat)
    m16 = _leaky(mx[0] + mx[1])

    loop_idx = jnp.arange(N, dtype=jnp.int32)
    pad_idx = N + jnp.arange(E_PAD - E_TOT, dtype=jnp.int32) % (NP - N)
    src = jnp.concatenate([edge_index[0].astype(jnp.int32), loop_idx,
                           pad_idx])
    dst = jnp.concatenate([edge_index[1].astype(jnp.int32), loop_idx,
                           pad_idx])
    ei = jnp.stack([src, dst]).reshape(2, NW * CH, K).transpose(1, 0, 2)

    pad_rows = NP - N
    a_s_p = jnp.concatenate([a_s, jnp.zeros((pad_rows, 16), jnp.float32)])
    a_d_p = jnp.concatenate([a_d, jnp.zeros((pad_rows, 16), jnp.float32)])
    he_p = jnp.concatenate([he, jnp.zeros((pad_rows, HE), jnp.bfloat16)])
    he_p = lax.bitcast_convert_type(
        he_p.reshape(NP, HE // 2, 2), jnp.int32)

    dpart = _sc1(ei, a_s_p, a_d_p, m16)
    denom = dpart[0] + dpart[1]

    addn = jnp.concatenate([a_d_p, denom], axis=1)
    epart, wpart = _sc2(ei, a_s_p, addn, he_p, m16)

    g = _tc_b1(h, wpart[:, :N, :], p16, W_pool, b_gat.reshape(1, HC),
               b_pool.reshape(1, D_HID))
    ebias = (b_gat @ W_emb + b_emb).reshape(1, D_EMB)
    emb, inf = _tc_b2(epart[:, :N, :], ebias, g, W_i1,
                      b_i1.reshape(1, 64), W_i2, b_i2.reshape(1, 1))
    return (emb, inf)


# speedup vs baseline: 66.7952x; 1.0158x over previous
import functools

import jax
import jax.numpy as jnp
from jax import lax
from jax.experimental import pallas as pl
from jax.experimental.pallas import tpu as pltpu
from jax.experimental.pallas import tpu_sc as plsc

N = 10000
E = 320000
D_IN = 128
H = 8
C = 128
HC = H * C
D_EMB = 32
HE = H * D_EMB
D_HID = 128

NC = 2
NS = 16
NW = NC * NS
K = 128
E_TOT = E + N
CH = 2 * (-(-E_TOT // (NW * K * 2)))
E_PAD = NW * K * CH
NP = 10112
NP_T = NP // NS

BLK = 1000


def _tc_a1_body(x_ref, wg_ref, asrc_ref, adst_ref,
                as_ref, ad_ref, mx_ref, fold_ref):
    i = pl.program_id(0)

    @pl.when(i == 0)
    def _():
        zs = jnp.zeros((D_IN, 8), dtype=jnp.float32)
        ts = wg_ref[...] * asrc_ref[...]
        td = wg_ref[...] * adst_ref[...]
        fold_ref[...] = jnp.concatenate(
            [jnp.sum(ts[:, hh * C:(hh + 1) * C], axis=1, keepdims=True)
             for hh in range(H)] + [zs]
            + [jnp.sum(td[:, hh * C:(hh + 1) * C], axis=1, keepdims=True)
               for hh in range(H)] + [zs], axis=1)

    xb = x_ref[...]
    asd = jnp.dot(xb, fold_ref[...], preferred_element_type=jnp.float32)
    a_s = asd[:, :16]
    a_d = asd[:, 16:]
    as_ref[...] = a_s
    ad_ref[...] = a_d
    ms = jnp.max(a_s, axis=0, keepdims=True)
    md = jnp.max(a_d, axis=0, keepdims=True)
    upd = jnp.concatenate([ms, md, jnp.full((6, 16), -jnp.inf, jnp.float32)],
                          axis=0)

    @pl.when(i == 0)
    def _():
        mx_ref[...] = upd

    @pl.when(i > 0)
    def _():
        mx_ref[...] = jnp.maximum(mx_ref[...], upd)


def _tc_a1(x, w_gat, asrc_flat, adst_flat):
    grid = N // BLK
    return pl.pallas_call(
        _tc_a1_body,
        grid=(grid,),
        in_specs=[
            pl.BlockSpec((BLK, D_IN), lambda i: (i, 0)),
            pl.BlockSpec((D_IN, HC), lambda i: (0, 0)),
            pl.BlockSpec((1, HC), lambda i: (0, 0)),
            pl.BlockSpec((1, HC), lambda i: (0, 0)),
        ],
        out_specs=[
            pl.BlockSpec((BLK, 16), lambda i: (i, 0)),
            pl.BlockSpec((BLK, 16), lambda i: (i, 0)),
            pl.BlockSpec((8, 16), lambda i: (0, 0)),
        ],
        out_shape=[
            jax.ShapeDtypeStruct((N, 16), jnp.float32),
            jax.ShapeDtypeStruct((N, 16), jnp.float32),
            jax.ShapeDtypeStruct((8, 16), jnp.float32),
        ],
        scratch_shapes=[pltpu.VMEM((D_IN, 32), jnp.float32)],
    )(x, w_gat, asrc_flat, adst_flat)


def _tc_a2_body(x_ref, wg_ref, bd_ref, he_ref, whe_ref):
    i = pl.program_id(0)

    @pl.when(i == 0)
    def _():
        whe_ref[...] = jnp.dot(wg_ref[...], bd_ref[...],
                               preferred_element_type=jnp.float32)

    he_ref[...] = jnp.dot(
        x_ref[...], whe_ref[...],
        preferred_element_type=jnp.float32).astype(jnp.bfloat16)


def _tc_a2(x, w_gat, bd):
    grid = N // BLK
    return pl.pallas_call(
        _tc_a2_body,
        grid=(grid,),
        in_specs=[
            pl.BlockSpec((BLK, D_IN), lambda i: (i, 0)),
            pl.BlockSpec((D_IN, HC), lambda i: (0, 0)),
            pl.BlockSpec((HC, HE), lambda i: (0, 0)),
        ],
        out_specs=pl.BlockSpec((BLK, HE), lambda i: (i, 0)),
        out_shape=jax.ShapeDtypeStruct((N, HE), jnp.bfloat16),
        scratch_shapes=[pltpu.VMEM((D_IN, HE), jnp.float32)],
    )(x, w_gat, bd)


def _leaky(t):
    return jnp.where(t >= 0.0, t, 0.2 * t)


ZCH = NP_T // 4


def _sc1_body(ei_hbm, as_hbm, ad_hbm, m_hbm, dpart_hbm,
              den_sh, i0, i1, i2, i3, as0, ad0, as1, ad1,
              s0, s1, z_v, m_v, semi, semg0, semg1, sems0, sems1):
    cid = lax.axis_index("c")
    sid = lax.axis_index("s")
    wid = cid * NS + sid

    @pl.loop(0, ZCH)
    def _(r):
        z_v[r] = jnp.zeros((16,), jnp.float32)
    for q in range(4):
        pltpu.sync_copy(z_v, den_sh.at[pl.ds(sid * NP_T + q * ZCH, ZCH)])
    pltpu.sync_copy(m_hbm, m_v)
    plsc.subcore_barrier()

    mvec = m_v[...]
    cb = wid * CH
    islot = (i0, i1, i2, i3)
    dat = ((as0, ad0), (as1, ad1))
    sv = (s0, s1)
    sgg = (semg0, semg1)
    sgs = (sems0, sems1)

    def fire_idx(j, q):
        pltpu.async_copy(ei_hbm.at[cb + j], islot[q], semi)

    def wait_idx(j, q):
        pltpu.make_async_copy(ei_hbm.at[cb + j], islot[q], semi).wait()

    def fire_g(q, b):
        pltpu.async_copy(as_hbm.at[islot[q].at[0]], dat[b][0], sgg[b])
        pltpu.async_copy(ad_hbm.at[islot[q].at[1]], dat[b][1], sgg[b])

    def wait_g(q, b):
        pltpu.make_async_copy(as_hbm.at[islot[q].at[0]], dat[b][0],
                              sgg[b]).wait()
        pltpu.make_async_copy(ad_hbm.at[islot[q].at[1]], dat[b][1],
                              sgg[b]).wait()

    def drain_scat(q, b):
        pltpu.make_async_copy(sv[b], den_sh.at[islot[q].at[1]],
                              sgs[b]).wait()

    fire_idx(0, 0)
    wait_idx(0, 0)
    fire_g(0, 0)
    fire_idx(1, 1)

    @pl.loop(0, CH // 2)
    def _(jo):
        for b in range(2):
            j = jo * 2 + b

            def body(q, qn, qp):
                wait_g(q, b)

                @pl.when(j >= 2)
                def _():
                    drain_scat(qp, b)

                @pl.when(j < CH - 1)
                def _():
                    wait_idx(j + 1, qn)
                    fire_g(qn, 1 - b)

                as_v, ad_v = dat[b]
                s_v = sv[b]

                @pl.loop(0, K)
                def _(r):
                    t = _leaky(as_v[r] + ad_v[r]) - mvec
                    s_v[r] = jnp.exp(t)

                pltpu.async_copy(s_v, den_sh.at[islot[q].at[1]], sgs[b],
                                 add=True)

                @pl.when(j < CH - 2)
                def _():
                    fire_idx(j + 2, qp)

            @pl.when(jo % 2 == 0)
            def _():
                body(b, b + 1 if b == 0 else 2, b + 2)

            @pl.when(jo % 2 == 1)
            def _():
                body(b + 2, 3 if b == 0 else 0, b)

    if CH % 4 == 0:
        drain_scat(2, 0)
        drain_scat(3, 1)
    else:
        drain_scat(0, 0)
        drain_scat(1, 1)

    plsc.subcore_barrier()
    pltpu.sync_copy(den_sh.at[pl.ds(sid * NP_T, NP_T)],
                    dpart_hbm.at[cid, pl.ds(sid * NP_T, NP_T)])


def _sc1(ei, a_s, a_d, m16):
    mesh = plsc.VectorSubcoreMesh(core_axis_name="c", subcore_axis_name="s", num_cores=NC, num_subcores=NS)
    f = pl.kernel(
        _sc1_body,
        out_type=jax.ShapeDtypeStruct((NC, NP, 16), jnp.float32),
        mesh=mesh,
        compiler_params=pltpu.CompilerParams(use_tc_tiling_on_sc=False),
        scratch_types=[
            pltpu.VMEM_SHARED((NP, 16), jnp.float32),
            pltpu.VMEM((2, K), jnp.int32),
            pltpu.VMEM((2, K), jnp.int32),
            pltpu.VMEM((2, K), jnp.int32),
            pltpu.VMEM((2, K), jnp.int32),
            pltpu.VMEM((K, 16), jnp.float32),
            pltpu.VMEM((K, 16), jnp.float32),
            pltpu.VMEM((K, 16), jnp.float32),
            pltpu.VMEM((K, 16), jnp.float32),
            pltpu.VMEM((K, 16), jnp.float32),
            pltpu.VMEM((K, 16), jnp.float32),
            pltpu.VMEM((ZCH, 16), jnp.float32),
            pltpu.VMEM((16,), jnp.float32),
            pltpu.SemaphoreType.DMA,
            pltpu.SemaphoreType.DMA,
            pltpu.SemaphoreType.DMA,
            pltpu.SemaphoreType.DMA,
            pltpu.SemaphoreType.DMA,
        ],
    )
    return f(ei, a_s, a_d, m16)


def _sc2_body(ei_hbm, as_hbm, addn_hbm, he_hbm, m_hbm,
              epart_hbm, wpart_hbm,
              emb_sh, w_sh, i0, i1, i2, i3,
              as0, ad0, he0, as1, ad1, he1,
              al0, msg0, al1, msg1, z_v, z16_v, m_v,
              semi, semg0, semg1, sems0, sems1):
    cid = lax.axis_index("c")
    sid = lax.axis_index("s")
    wid = cid * NS + sid

    @pl.loop(0, ZCH)
    def _(r):
        z_v[r] = jnp.zeros((32,), jnp.float32)
        z16_v[r] = jnp.zeros((16,), jnp.float32)
    for q in range(4):
        pltpu.sync_copy(z_v, emb_sh.at[pl.ds(sid * NP_T + q * ZCH, ZCH)])
        pltpu.sync_copy(z16_v, w_sh.at[pl.ds(sid * NP_T + q * ZCH, ZCH)])
    pltpu.sync_copy(m_hbm, m_v)
    plsc.subcore_barrier()

    mvec = m_v[...]
    cb = wid * CH
    islot = (i0, i1, i2, i3)
    dat = ((as0, ad0, he0), (as1, ad1, he1))
    sv = ((al0, msg0), (al1, msg1))
    sgg = (semg0, semg1)
    sgs = (sems0, sems1)

    def fire_idx(j, q):
        pltpu.async_copy(ei_hbm.at[cb + j], islot[q], semi)

    def wait_idx(j, q):
        pltpu.make_async_copy(ei_hbm.at[cb + j], islot[q], semi).wait()

    def fire_g(q, b):
        pltpu.async_copy(as_hbm.at[islot[q].at[0]], dat[b][0], sgg[b])
        pltpu.async_copy(addn_hbm.at[islot[q].at[1]], dat[b][1], sgg[b])
        pltpu.async_copy(he_hbm.at[islot[q].at[0]], dat[b][2], sgg[b])

    def wait_g(q, b):
        pltpu.make_async_copy(as_hbm.at[islot[q].at[0]], dat[b][0],
                              sgg[b]).wait()
        pltpu.make_async_copy(addn_hbm.at[islot[q].at[1]], dat[b][1],
                              sgg[b]).wait()
        pltpu.make_async_copy(he_hbm.at[islot[q].at[0]], dat[b][2],
                              sgg[b]).wait()

    def drain_scat(q, b):
        pltpu.make_async_copy(sv[b][1], emb_sh.at[islot[q].at[1]],
                              sgs[b]).wait()
        pltpu.make_async_copy(sv[b][0], w_sh.at[islot[q].at[0]],
                              sgs[b]).wait()

    fire_idx(0, 0)
    wait_idx(0, 0)
    fire_g(0, 0)
    fire_idx(1, 1)

    @pl.loop(0, CH // 2)
    def _(jo):
        for b in range(2):
            j = jo * 2 + b

            def body(q, qn, qp):
                wait_g(q, b)

                @pl.when(j >= 2)
                def _():
                    drain_scat(qp, b)

                @pl.when(j < CH - 1)
                def _():
                    wait_idx(j + 1, qn)
                    fire_g(qn, 1 - b)

                as_v, ad_v, he_v = dat[b]
                al_v, msg_v = sv[b]

                @pl.loop(0, K)
                def _(r):
                    t = _leaky(as_v[r] + ad_v[r, pl.ds(0, 16)]) - mvec
                    al = jnp.exp(t) / ad_v[r, pl.ds(16, 16)]
                    al_v[r] = al
                    acc0 = jnp.zeros((16,), jnp.float32)
                    acc1 = jnp.zeros((16,), jnp.float32)
                    for hh in range(H):
                        a = al[hh]
                        u = he_v[r, pl.ds(hh * 16, 16)]
                        va = lax.bitcast_convert_type(u << 16, jnp.float32)
                        vb = lax.bitcast_convert_type(
                            u & jnp.int32(-65536), jnp.float32)
                        acc0 = acc0 + a * va
                        acc1 = acc1 + a * vb
                    msg_v[r, pl.ds(0, 16)] = acc0
                    msg_v[r, pl.ds(16, 16)] = acc1

                pltpu.async_copy(msg_v, emb_sh.at[islot[q].at[1]], sgs[b],
                                 add=True)
                pltpu.async_copy(al_v, w_sh.at[islot[q].at[0]], sgs[b],
                                 add=True)

                @pl.when(j < CH - 2)
                def _():
                    fire_idx(j + 2, qp)

            @pl.when(jo % 2 == 0)
            def _():
                body(b, b + 1 if b == 0 else 2, b + 2)

            @pl.when(jo % 2 == 1)
            def _():
                body(b + 2, 3 if b == 0 else 0, b)

    if CH % 4 == 0:
        drain_scat(2, 0)
        drain_scat(3, 1)
    else:
        drain_scat(0, 0)
        drain_scat(1, 1)

    plsc.subcore_barrier()
    pltpu.sync_copy(emb_sh.at[pl.ds(sid * NP_T, NP_T)],
                    epart_hbm.at[cid, pl.ds(sid * NP_T, NP_T)])
    pltpu.sync_copy(w_sh.at[pl.ds(sid * NP_T, NP_T)],
                    wpart_hbm.at[cid, pl.ds(sid * NP_T, NP_T)])


def _sc2(ei, a_s, addn, he, m16):
    mesh = plsc.VectorSubcoreMesh(core_axis_name="c", subcore_axis_name="s", num_cores=NC, num_subcores=NS)
    f = pl.kernel(
        _sc2_body,
        out_type=(jax.ShapeDtypeStruct((NC, NP, 32), jnp.float32),
                  jax.ShapeDtypeStruct((NC, NP, 16), jnp.float32)),
        mesh=mesh,
        compiler_params=pltpu.CompilerParams(use_tc_tiling_on_sc=False),
        scratch_types=[
            pltpu.VMEM_SHARED((NP, 32), jnp.float32),
            pltpu.VMEM_SHARED((NP, 16), jnp.float32),
            pltpu.VMEM((2, K), jnp.int32),
            pltpu.VMEM((2, K), jnp.int32),
            pltpu.VMEM((2, K), jnp.int32),
            pltpu.VMEM((2, K), jnp.int32),
            pltpu.VMEM((K, 16), jnp.float32),
            pltpu.VMEM((K, 32), jnp.float32),
            pltpu.VMEM((K, HE // 2), jnp.int32),
            pltpu.VMEM((K, 16), jnp.float32),
            pltpu.VMEM((K, 32), jnp.float32),
            pltpu.VMEM((K, HE // 2), jnp.int32),
            pltpu.VMEM((K, 16), jnp.float32),
            pltpu.VMEM((K, 32), jnp.float32),
            pltpu.VMEM((K, 16), jnp.float32),
            pltpu.VMEM((K, 32), jnp.float32),
            pltpu.VMEM((ZCH, 32), jnp.float32),
            pltpu.VMEM((ZCH, 16), jnp.float32),
            pltpu.VMEM((16,), jnp.float32),
            pltpu.SemaphoreType.DMA,
            pltpu.SemaphoreType.DMA,
            pltpu.SemaphoreType.DMA,
            pltpu.SemaphoreType.DMA,
            pltpu.SemaphoreType.DMA,
        ],
    )
    return f(ei, a_s, addn, he, m16)


def _tc_b1_body(x_ref, wg_ref, wp_ref, p16_ref, wpool_ref, bg_ref, bp_ref,
                g_ref, acc_ref):
    i = pl.program_id(0)
    hb = jnp.dot(x_ref[...], wg_ref[...], preferred_element_type=jnp.float32)
    w_blk = wp_ref[0] + wp_ref[1]
    wexp = jnp.dot(w_blk, p16_ref[...],
                   preferred_element_type=jnp.float32)
    part = jnp.sum(hb * wexp, axis=0, keepdims=True)

    @pl.when(i == 0)
    def _():
        acc_ref[...] = part

    @pl.when(i > 0)
    def _():
        acc_ref[...] = acc_ref[...] + part

    @pl.when(i == pl.num_programs(0) - 1)
    def _():
        m = acc_ref[...] / jnp.float32(N) + bg_ref[...]
        g_ref[...] = jnp.maximum(
            jnp.dot(m, wpool_ref[...], preferred_element_type=jnp.float32)
            + bp_ref[...], 0.0)


def _tc_b1(x, w_gat, wpart, p16, w_pool, b_gat, b_pool):
    grid = N // BLK
    return pl.pallas_call(
        _tc_b1_body,
        grid=(grid,),
        in_specs=[
            pl.BlockSpec((BLK, D_IN), lambda i: (i, 0)),
            pl.BlockSpec((D_IN, HC), lambda i: (0, 0)),
            pl.BlockSpec((NC, BLK, 16), lambda i: (0, i, 0)),
            pl.BlockSpec((16, HC), lambda i: (0, 0)),
            pl.BlockSpec((HC, D_HID), lambda i: (0, 0)),
            pl.BlockSpec((1, HC), lambda i: (0, 0)),
            pl.BlockSpec((1, D_HID), lambda i: (0, 0)),
        ],
        out_specs=pl.BlockSpec((1, D_HID), lambda i: (0, 0)),
        out_shape=jax.ShapeDtypeStruct((1, D_HID), jnp.float32),
        scratch_shapes=[pltpu.VMEM((1, HC), jnp.float32)],
    )(x, w_gat, wpart, p16, w_pool, b_gat, b_pool)


def _tc_b2_body(ep_ref, ebias_ref, g_ref, wi1_ref, bi1_ref, wi2_ref, bi2_ref,
                emb_ref, inf_ref):
    emb = ep_ref[0] + ep_ref[1] + ebias_ref[...]
    emb_ref[...] = emb
    gb = jnp.broadcast_to(g_ref[...], (BLK, D_HID))
    comb = jnp.concatenate([emb, gb], axis=1)
    hid = jnp.maximum(
        jnp.dot(comb, wi1_ref[...], preferred_element_type=jnp.float32)
        + bi1_ref[...], 0.0)
    z = jnp.dot(hid, wi2_ref[...], preferred_element_type=jnp.float32) \
        + bi2_ref[...]
    inf_ref[...] = 1.0 / (1.0 + jnp.exp(-z))


def _tc_b2(epart, ebias, g, w_i1, b_i1, w_i2, b_i2):
    grid = N // BLK
    return pl.pallas_call(
        _tc_b2_body,
        grid=(grid,),
        in_specs=[
            pl.BlockSpec((NC, BLK, D_EMB), lambda i: (0, i, 0)),
            pl.BlockSpec((1, D_EMB), lambda i: (0, 0)),
            pl.BlockSpec((1, D_HID), lambda i: (0, 0)),
            pl.BlockSpec((D_EMB + D_HID, 64), lambda i: (0, 0)),
            pl.BlockSpec((1, 64), lambda i: (0, 0)),
            pl.BlockSpec((64, 1), lambda i: (0, 0)),
            pl.BlockSpec((1, 1), lambda i: (0, 0)),
        ],
        out_specs=[
            pl.BlockSpec((BLK, D_EMB), lambda i: (i, 0)),
            pl.BlockSpec((BLK, 1), lambda i: (i, 0)),
        ],
        out_shape=[
            jax.ShapeDtypeStruct((N, D_EMB), jnp.float32),
            jax.ShapeDtypeStruct((N, 1), jnp.float32),
        ],
    )(epart, ebias, g, w_i1, b_i1, w_i2, b_i2)


def kernel(x, edge_index, W_gat, att_src, att_dst, b_gat, W_pool, b_pool,
           W_emb, b_emb, W_i1, b_i1, W_i2, b_i2):
    asrc_flat = att_src.reshape(1, HC)
    adst_flat = att_dst.reshape(1, HC)
    bd = jnp.zeros((H, C, H, D_EMB), jnp.float32)
    ii = jnp.arange(H)
    bd = bd.at[ii, :, ii, :].set(W_emb.reshape(H, C, D_EMB))
    bd = bd.reshape(HC, HE)
    blk_perm = jnp.stack(
        [jnp.arange(16), jnp.arange(16, 32)], axis=1).reshape(32)
    perm_cols = (jnp.arange(H)[:, None] * 32 + blk_perm[None, :]).reshape(HE)
    bd = bd[:, perm_cols]
    p16 = jnp.zeros((16, H, C), jnp.float32)
    p16 = p16.at[jnp.arange(H), jnp.arange(H), :].set(1.0)
    p16 = p16.reshape(16, HC)

    a_s, a_d, mx = _tc_a1(x, W_gat, asrc_flat, adst_flat)
    m16 = _leaky(mx[0] + mx[1])

    loop_idx = jnp.arange(N, dtype=jnp.int32)
    pad_idx = N + jnp.arange(E_PAD - E_TOT, dtype=jnp.int32) % (NP - N)
    src = jnp.concatenate([edge_index[0].astype(jnp.int32), loop_idx,
                           pad_idx])
    dst = jnp.concatenate([edge_index[1].astype(jnp.int32), loop_idx,
                           pad_idx])
    ei = jnp.stack([src, dst]).reshape(2, NW * CH, K).transpose(1, 0, 2)

    pad_rows = NP - N
    a_s_p = jnp.concatenate([a_s, jnp.zeros((pad_rows, 16), jnp.float32)])
    a_d_p = jnp.concatenate([a_d, jnp.zeros((pad_rows, 16), jnp.float32)])

    dpart = _sc1(ei, a_s_p, a_d_p, m16)

    he = _tc_a2(x, W_gat, bd)
    he_p = jnp.concatenate([he, jnp.zeros((pad_rows, HE), jnp.bfloat16)])
    he_p = lax.bitcast_convert_type(
        he_p.reshape(NP, HE // 2, 2), jnp.int32)

    denom = dpart[0] + dpart[1]

    addn = jnp.concatenate([a_d_p, denom], axis=1)
    epart, wpart = _sc2(ei, a_s_p, addn, he_p, m16)

    g = _tc_b1(x, W_gat, wpart[:, :N, :], p16, W_pool, b_gat.reshape(1, HC),
               b_pool.reshape(1, D_HID))
    ebias = (b_gat @ W_emb + b_emb).reshape(1, D_EMB)
    emb, inf = _tc_b2(epart[:, :N, :], ebias, g, W_i1,
                      b_i1.reshape(1, 64), W_i2, b_i2.reshape(1, 1))
    return (emb, inf)
